# Initial kernel scaffold; baseline (speedup 1.0000x reference)
import jax
import jax.numpy as jnp
from jax.experimental import pallas as pl


def kernel(x, pos, edge_index, params):
    raise NotImplementedError("write your pallas kernel here")


# speedup vs baseline: 1.2602x; 1.2602x over previous
import functools
import math

import jax
import jax.numpy as jnp
from jax.experimental import pallas as pl
from jax.experimental.pallas import tpu as pltpu

HID = 128
EPS = 1e-5


def _ln(z, g, b):
    mu = jnp.mean(z, axis=-1, keepdims=True)
    var = jnp.mean((z - mu) ** 2, axis=-1, keepdims=True)
    return (z - mu) * jax.lax.rsqrt(var + EPS) * g + b


def _node_kernel(x_ref, wf_ref, bf_ref, gf_ref, b2f_ref, wwf_ref, wq_ref,
                 bq_ref, fw_ref, qn_ref):
    x = x_ref[...]
    z = jnp.maximum(
        jnp.dot(x, wf_ref[...], preferred_element_type=jnp.float32)
        + bf_ref[...], 0.0)
    feat = _ln(z, gf_ref[...], b2f_ref[...])
    fw_ref[...] = jnp.dot(feat, wwf_ref[...], preferred_element_type=jnp.float32)
    qn_ref[...] = (jnp.dot(feat, wq_ref[...], preferred_element_type=jnp.float32)
                   + bq_ref[...])


def _node_precompute(x, p):
    n = x.shape[0]
    blk = 512
    grid = (pl.cdiv(n, blk),)
    full = lambda s: pl.BlockSpec(s, lambda i: (0, 0))
    fw, qn = pl.pallas_call(
        _node_kernel,
        grid=grid,
        in_specs=[
            pl.BlockSpec((blk, HID), lambda i: (i, 0)),
            full((HID, HID)),
            full((1, HID)), full((1, HID)), full((1, HID)),
            full((HID, HID)),
            full((HID, HID)),
            full((1, HID)),
        ],
        out_specs=[
            pl.BlockSpec((blk, HID), lambda i: (i, 0)),
            pl.BlockSpec((blk, HID), lambda i: (i, 0)),
        ],
        out_shape=[
            jax.ShapeDtypeStruct((n, HID), jnp.float32),
            jax.ShapeDtypeStruct((n, HID), jnp.float32),
        ],
    )(x, p['Wf'], p['bf'][None], p['gf'][None], p['b2f'][None],
      p['Ww'][:HID], p['Wq'], p['bq'][None])
    return fw, qn


def _edge_kernel(dfw_ref, qd_ref, dp_ref, wwp_ref, bw_ref, gw_ref, b2w_ref,
                 wp_ref, bp_ref, gp_ref, b2p_ref, wk_ref, bk_ref,
                 w_ref, score_ref):
    dp = dp_ref[...]
    h = (dfw_ref[...]
         + jnp.dot(dp, wwp_ref[...], preferred_element_type=jnp.float32)
         + bw_ref[...])
    W = _ln(jnp.maximum(h, 0.0), gw_ref[...], b2w_ref[...])
    pe = _ln(jnp.maximum(
        jnp.dot(dp, wp_ref[...], preferred_element_type=jnp.float32)
        + bp_ref[...], 0.0), gp_ref[...], b2p_ref[...])
    q = qd_ref[...] + pe
    k = jnp.dot(W, wk_ref[...], preferred_element_type=jnp.float32) + bk_ref[...]
    w_ref[...] = W
    score_ref[...] = jnp.sum(q * k, axis=-1) * (1.0 / math.sqrt(float(HID)))


def _edge_dense(dfw, qd, dp8, p):
    e = dfw.shape[0]
    blk = 1024
    grid = (pl.cdiv(e, blk),)
    full = lambda s: pl.BlockSpec(s, lambda i: tuple(0 for _ in s))
    wwp8 = jnp.zeros((8, HID), jnp.float32).at[:3].set(p['Ww'][HID:])
    wp8 = jnp.zeros((8, HID), jnp.float32).at[:3].set(p['Wp'])
    W, score = pl.pallas_call(
        _edge_kernel,
        grid=grid,
        in_specs=[
            pl.BlockSpec((blk, HID), lambda i: (i, 0)),
            pl.BlockSpec((blk, HID), lambda i: (i, 0)),
            pl.BlockSpec((blk, 8), lambda i: (i, 0)),
            full((8, HID)),
            full((1, HID)), full((1, HID)), full((1, HID)),
            full((8, HID)),
            full((1, HID)), full((1, HID)), full((1, HID)),
            full((HID, HID)),
            full((1, HID)),
        ],
        out_specs=[
            pl.BlockSpec((blk, HID), lambda i: (i, 0)),
            pl.BlockSpec((blk,), lambda i: (i,)),
        ],
        out_shape=[
            jax.ShapeDtypeStruct((e, HID), jnp.float32),
            jax.ShapeDtypeStruct((e,), jnp.float32),
        ],
    )(dfw, qd, dp8, wwp8, p['bw'][None], p['gw'][None], p['b2w'][None],
      wp8, p['bp'][None], p['gp'][None], p['b2p'][None],
      p['Wk'], p['bk'][None])
    return W, score


def _final_kernel(s_ref, d_ref, x_ref, gn_ref, bn_ref, out_ref):
    agg = s_ref[...] / (d_ref[...][:, None] + 1e-16)
    out_ref[...] = _ln(agg + x_ref[...], gn_ref[...], bn_ref[...])


def _final_ln(S, d, x, p):
    n = x.shape[0]
    blk = 512
    grid = (pl.cdiv(n, blk),)
    full = lambda s: pl.BlockSpec(s, lambda i: (0, 0))
    return pl.pallas_call(
        _final_kernel,
        grid=grid,
        in_specs=[
            pl.BlockSpec((blk, HID), lambda i: (i, 0)),
            pl.BlockSpec((blk,), lambda i: (i,)),
            pl.BlockSpec((blk, HID), lambda i: (i, 0)),
            full((1, HID)), full((1, HID)),
        ],
        out_specs=pl.BlockSpec((blk, HID), lambda i: (i, 0)),
        out_shape=jax.ShapeDtypeStruct((n, HID), jnp.float32),
    )(S, d, x, p['gn'][None], p['bn'][None])


def _block(x, pos8, src, dst, p):
    n = x.shape[0]
    fw, qn = _node_precompute(x, p)
    dfw = fw[dst] - fw[src]
    qd = qn[dst]
    dp8 = pos8[dst] - pos8[src]
    W, score = _edge_dense(dfw, qd, dp8, p)
    m = jax.ops.segment_max(score, dst, num_segments=n)
    m = jnp.where(jnp.isfinite(m), m, 0.0)
    e = jnp.exp(score - m[dst])
    d = jax.ops.segment_sum(e, dst, num_segments=n)
    S = jax.ops.segment_sum(e[:, None] * W, dst, num_segments=n)
    return _final_ln(S, d, x, p)


def kernel(x, pos, edge_index, params):
    src = edge_index[0]
    dst = edge_index[1]
    pos8 = jnp.zeros((pos.shape[0], 8), jnp.float32).at[:, :3].set(pos)
    for p in params:
        x = _block(x, pos8, src, dst, p)
    return x


# speedup vs baseline: 5.4339x; 4.3119x over previous
import functools
import math

import jax
import jax.numpy as jnp
from jax import lax
from jax.experimental import pallas as pl
from jax.experimental.pallas import tpu as pltpu
from jax.experimental.pallas import tpu_sc as plsc

HID = 128
EPS = 1e-5
TW = 144
NC = 2
NS = 16
L = 16
NW = NC * NS


def _ln(z, g, b):
    mu = jnp.mean(z, axis=-1, keepdims=True)
    var = jnp.mean((z - mu) ** 2, axis=-1, keepdims=True)
    return (z - mu) * jax.lax.rsqrt(var + EPS) * g + b


def _vgather(x, idx):
    return lax.gather(
        x, idx[:, None],
        lax.GatherDimensionNumbers(
            offset_dims=(), collapsed_slice_dims=(0,), start_index_map=(0,)),
        (1,), mode=lax.GatherScatterMode.PROMISE_IN_BOUNDS)


def _node_body(x_ref, wf_ref, bf_ref, gf_ref, b2f_ref, feat_ref):
    z = jnp.maximum(
        jnp.dot(x_ref[...], wf_ref[...], preferred_element_type=jnp.float32)
        + bf_ref[...], 0.0)
    feat_ref[...] = _ln(z, gf_ref[...], b2f_ref[...])


def _node_precompute(x, p):
    n = x.shape[0]
    blk = 512
    full = lambda s: pl.BlockSpec(s, lambda i: (0, 0))
    return pl.pallas_call(
        _node_body,
        grid=(pl.cdiv(n, blk),),
        in_specs=[
            pl.BlockSpec((blk, HID), lambda i: (i, 0)),
            full((HID, HID)),
            full((1, HID)), full((1, HID)), full((1, HID)),
        ],
        out_specs=pl.BlockSpec((blk, HID), lambda i: (i, 0)),
        out_shape=jax.ShapeDtypeStruct((n, HID), jnp.float32),
    )(x, p['Wf'], p['bf'][None], p['gf'][None], p['b2f'][None])


@functools.cache
def _make_gather(n, e):
    epw = e // NW
    CG = 80
    nch = epw // CG
    rp_a = (n // NS) // 8 * 8
    rp_last = n - (NS - 1) * rp_a
    mesh = plsc.VectorSubcoreMesh(core_axis_name="c", subcore_axis_name="s")

    @functools.partial(
        pl.kernel,
        out_type=[jax.ShapeDtypeStruct((e, HID), jnp.float32),
                  jax.ShapeDtypeStruct((e, HID), jnp.float32),
                  jax.ShapeDtypeStruct((e,), jnp.float32),
                  jax.ShapeDtypeStruct((e,), jnp.float32),
                  jax.ShapeDtypeStruct((e,), jnp.float32)],
        mesh=mesh,
        compiler_params=pltpu.CompilerParams(needs_layout_passes=False),
        scratch_types=[
            pltpu.VMEM((3 * n,), jnp.float32),
            pltpu.VMEM((CG,), jnp.int32),
            pltpu.VMEM((CG,), jnp.int32),
            pltpu.VMEM((CG, HID), jnp.float32),
            pltpu.VMEM((CG, HID), jnp.float32),
            pltpu.VMEM((3, CG), jnp.float32),
            pltpu.SemaphoreType.DMA,
            pltpu.SemaphoreType.DMA,
        ],
    )
    def gather_k(tab_hbm, pos3_hbm, src_hbm, dst_hbm,
                 gd_out, gs_out, dpx_out, dpy_out, dpz_out,
                 pos3_v, idxd_v, idxs_v, gd_v, gs_v, dp3_v,
                 sem1, sem2):
        c = lax.axis_index("c")
        s = lax.axis_index("s")
        wid = s * NC + c
        pltpu.sync_copy(pos3_hbm, pos3_v)
        base_w = wid * epw
        dp_outs = (dpx_out, dpy_out, dpz_out)

        def chunk(i, carry):
            base = base_w + i * CG
            pltpu.sync_copy(dst_hbm.at[pl.ds(base, CG)], idxd_v)
            pltpu.sync_copy(src_hbm.at[pl.ds(base, CG)], idxs_v)
            cp1 = pltpu.async_copy(tab_hbm.at[idxd_v], gd_v, sem1)
            cp2 = pltpu.async_copy(tab_hbm.at[idxs_v], gs_v, sem2)

            def vec(v, carry2):
                ivd = idxd_v[pl.ds(v * L, L)]
                ivs = idxs_v[pl.ds(v * L, L)]
                for comp in range(3):
                    off = jnp.int32(comp * n)
                    d = (plsc.load_gather(pos3_v, [ivd + off])
                         - plsc.load_gather(pos3_v, [ivs + off]))
                    dp3_v[comp, pl.ds(v * L, L)] = d
                return carry2

            lax.fori_loop(0, CG // L, vec, 0)
            cp1.wait()
            cp2.wait()
            pltpu.sync_copy(gd_v, gd_out.at[pl.ds(base, CG)])
            pltpu.sync_copy(gs_v, gs_out.at[pl.ds(base, CG)])
            for comp in range(3):
                pltpu.sync_copy(dp3_v.at[comp], dp_outs[comp].at[pl.ds(base, CG)])
            return carry

        lax.fori_loop(0, nch, chunk, 0)

    return gather_k


def _edge_body(gd_ref, gs_ref, dpx_ref, dpy_ref, dpz_ref,
               wwf_ref, wwp_ref, bw_ref, gw_ref, b2w_ref,
               wp_ref, bp_ref, gp_ref, b2p_ref, wq_ref, bq_ref,
               wk_ref, bk_ref, w_ref, score_ref):
    gd = gd_ref[...]
    gs = gs_ref[...]
    dpc = (dpx_ref[...][:, None], dpy_ref[...][:, None], dpz_ref[...][:, None])
    wwp = wwp_ref[...]
    wp = wp_ref[...]
    tdp = dpc[0] * wwp[0:1] + dpc[1] * wwp[1:2] + dpc[2] * wwp[2:3]
    pdp = dpc[0] * wp[0:1] + dpc[1] * wp[1:2] + dpc[2] * wp[2:3]
    h = (jnp.dot(gd - gs, wwf_ref[...], preferred_element_type=jnp.float32)
         + tdp + bw_ref[...])
    W = _ln(jnp.maximum(h, 0.0), gw_ref[...], b2w_ref[...])
    pe = _ln(jnp.maximum(pdp + bp_ref[...], 0.0), gp_ref[...], b2p_ref[...])
    q = (jnp.dot(gd, wq_ref[...], preferred_element_type=jnp.float32)
         + bq_ref[...] + pe)
    k = jnp.dot(W, wk_ref[...], preferred_element_type=jnp.float32) + bk_ref[...]
    w_ref[...] = W
    score_ref[...] = jnp.sum(q * k, axis=-1) * (1.0 / math.sqrt(float(HID)))


def _edge_dense(gd, gs, dpx, dpy, dpz, p):
    e = gd.shape[0]
    blk = 1024
    full = lambda s: pl.BlockSpec(s, lambda i: (0, 0))
    return pl.pallas_call(
        _edge_body,
        grid=(pl.cdiv(e, blk),),
        in_specs=[
            pl.BlockSpec((blk, HID), lambda i: (i, 0)),
            pl.BlockSpec((blk, HID), lambda i: (i, 0)),
            pl.BlockSpec((blk,), lambda i: (i,)),
            pl.BlockSpec((blk,), lambda i: (i,)),
            pl.BlockSpec((blk,), lambda i: (i,)),
            full((HID, HID)),
            full((8, HID)),
            full((1, HID)), full((1, HID)), full((1, HID)),
            full((8, HID)),
            full((1, HID)), full((1, HID)), full((1, HID)),
            full((HID, HID)),
            full((1, HID)),
            full((HID, HID)),
            full((1, HID)),
        ],
        out_specs=[
            pl.BlockSpec((blk, HID), lambda i: (i, 0)),
            pl.BlockSpec((blk,), lambda i: (i,)),
        ],
        out_shape=[
            jax.ShapeDtypeStruct((e, HID), jnp.float32),
            jax.ShapeDtypeStruct((e,), jnp.float32),
        ],
    )(gd, gs, dpx, dpy, dpz,
      p['Ww'][:HID],
      jnp.zeros((8, HID), jnp.float32).at[:3].set(p['Ww'][HID:]),
      p['bw'][None], p['gw'][None], p['b2w'][None],
      jnp.zeros((8, HID), jnp.float32).at[:3].set(p['Wp']),
      p['bp'][None], p['gp'][None], p['b2p'][None],
      p['Wq'], p['bq'][None], p['Wk'], p['bk'][None])


@functools.cache
def _make_segmax(n, e):
    epw = e // NW
    CH = 2000
    nch = epw // CH
    nvec = CH // L
    mesh = plsc.VectorSubcoreMesh(core_axis_name="c", subcore_axis_name="s")

    @functools.partial(
        pl.kernel,
        out_type=jax.ShapeDtypeStruct((NW, n), jnp.float32),
        mesh=mesh,
        compiler_params=pltpu.CompilerParams(needs_layout_passes=False),
        scratch_types=[
            pltpu.VMEM((n,), jnp.float32),
            pltpu.VMEM((CH,), jnp.float32),
            pltpu.VMEM((CH,), jnp.int32),
        ],
    )
    def segmax_k(score_hbm, dst_hbm, m_out, m_v, sc_v, id_v):
        c = lax.axis_index("c")
        s = lax.axis_index("s")
        wid = s * NC + c
        neg = jnp.full((L,), -jnp.inf, jnp.float32)

        def zi(j, carry):
            m_v[pl.ds(j * L, L)] = neg
            return carry

        lax.fori_loop(0, n // L, zi, 0)
        base_w = wid * epw

        def chunk(i, carry):
            base = base_w + i * CH
            pltpu.sync_copy(score_hbm.at[pl.ds(base, CH)], sc_v)
            pltpu.sync_copy(dst_hbm.at[pl.ds(base, CH)], id_v)

            def vec(v, carry2):
                iv = id_v[pl.ds(v * L, L)]
                sv = sc_v[pl.ds(v * L, L)]
                ks, vs = plsc.sort_key_val(iv, sv)
                iota = lax.iota(jnp.int32, L)
                for sh in (1, 2, 4, 8):
                    pidx = jnp.maximum(iota - sh, 0)
                    valid = (iota >= sh) & (_vgather(ks, pidx) == ks)
                    vs = jnp.maximum(
                        vs, jnp.where(valid, _vgather(vs, pidx), -jnp.inf))
                is_last = (iota == L - 1) | (
                    _vgather(ks, jnp.minimum(iota + 1, L - 1)) != ks)
                cur = plsc.load_gather(m_v, [ks])
                upd = is_last & (vs > cur)
                plsc.store_scatter(m_v, [ks], vs, mask=upd)
                return carry2

            lax.fori_loop(0, nvec, vec, 0)
            return carry

        lax.fori_loop(0, nch, chunk, 0)
        pltpu.sync_copy(m_v, m_out.at[wid])

    return segmax_k


def _mreduce_body(mp_ref, m_ref):
    m_ref[...] = jnp.max(mp_ref[...], axis=0)


def _mreduce(m_part):
    n = m_part.shape[1]
    nb = 1024
    return pl.pallas_call(
        _mreduce_body,
        grid=(pl.cdiv(n, nb),),
        in_specs=[pl.BlockSpec((NW, nb), lambda i: (0, i))],
        out_specs=pl.BlockSpec((nb,), lambda i: (i,)),
        out_shape=jax.ShapeDtypeStruct((n,), jnp.float32),
    )(m_part)


@functools.cache
def _make_scatter(n, e):
    epw = e // NW
    CH = 80
    SUB = 80
    nsub = CH // SUB
    nch = epw // CH
    vps = SUB // L
    rp_a = (n // NS) // 8 * 8
    rp_last = n - (NS - 1) * rp_a
    mesh = plsc.VectorSubcoreMesh(core_axis_name="c", subcore_axis_name="s")

    @functools.partial(
        pl.kernel,
        out_type=[jax.ShapeDtypeStruct((NC, n, HID), jnp.float32),
                  jax.ShapeDtypeStruct((NW, n), jnp.float32)],
        mesh=mesh,
        compiler_params=pltpu.CompilerParams(needs_layout_passes=False),
        scratch_types=[
            pltpu.VMEM_SHARED((n, HID), jnp.float32),
            pltpu.VMEM((n,), jnp.float32),
            pltpu.VMEM((n,), jnp.float32),
            pltpu.VMEM((CH, HID), jnp.float32),
            pltpu.VMEM((nsub, SUB), jnp.int32),
            pltpu.VMEM((CH,), jnp.float32),
        ],
    )
    def scat_k(w_hbm, score_hbm, dst_hbm, m_hbm, s_out, d_out,
               S_sh, m_v, d_v, w_v, id_v, sc_v):
        c = lax.axis_index("c")
        s = lax.axis_index("s")
        wid = s * NC + c
        pltpu.sync_copy(m_hbm, m_v)
        zf = jnp.zeros((L,), jnp.float32)

        def zd(j, cy):
            d_v[pl.ds(j * L, L)] = zf
            return cy

        lax.fori_loop(0, n // L, zd, 0)

        def zw(j, cy):
            for k8 in range(HID // L):
                w_v[j, pl.ds(k8 * L, L)] = zf
            return cy

        lax.fori_loop(0, CH, zw, 0)
        my_base = s * rp_a
        nz_full = rp_a // CH
        nz_rem = rp_a - nz_full * CH
        nz_full_last = rp_last // CH
        nz_rem_last = rp_last - nz_full_last * CH

        def zcopy(j, cy):
            pltpu.sync_copy(w_v, S_sh.at[pl.ds(my_base + j * CH, CH)])
            return cy

        @pl.when(s < NS - 1)
        def _():
            lax.fori_loop(0, nz_full, zcopy, 0)
            if nz_rem:
                pltpu.sync_copy(
                    w_v.at[pl.ds(0, nz_rem)],
                    S_sh.at[pl.ds(my_base + nz_full * CH, nz_rem)])

        @pl.when(s == NS - 1)
        def _():
            lax.fori_loop(0, nz_full_last, zcopy, 0)
            if nz_rem_last:
                pltpu.sync_copy(
                    w_v.at[pl.ds(0, nz_rem_last)],
                    S_sh.at[pl.ds(my_base + nz_full_last * CH, nz_rem_last)])

        plsc.subcore_barrier()
        base_w = wid * epw

        def chunk(i, cy):
            base = base_w + i * CH
            pltpu.sync_copy(score_hbm.at[pl.ds(base, CH)], sc_v)

            def sub_idx(j, cy2):
                pltpu.sync_copy(dst_hbm.at[pl.ds(base + j * SUB, SUB)],
                                id_v.at[j])
                return cy2

            lax.fori_loop(0, nsub, sub_idx, 0)
            pltpu.sync_copy(w_hbm.at[pl.ds(base, CH)], w_v)

            def subloop(j, cy2):
                def vec(u, cy3):
                    off = j * SUB + u * L
                    iv = id_v[j, pl.ds(u * L, L)]
                    sv = sc_v[pl.ds(off, L)]
                    mg = plsc.load_gather(m_v, [iv])
                    ev = jnp.exp(sv - mg)
                    ks, vs = plsc.sort_key_val(iv, ev)
                    iota = lax.iota(jnp.int32, L)
                    for sh in (1, 2, 4, 8):
                        pidx = jnp.maximum(iota - sh, 0)
                        valid = (iota >= sh) & (_vgather(ks, pidx) == ks)
                        vs = vs + jnp.where(valid, _vgather(vs, pidx), 0.0)
                    is_last = (iota == L - 1) | (
                        _vgather(ks, jnp.minimum(iota + 1, L - 1)) != ks)
                    plsc.addupdate_scatter(d_v, [ks], vs, mask=is_last)

                    def rowscale(r, cy4):
                        eb = _vgather(ev, jnp.full((L,), r, jnp.int32))
                        row = off + r
                        for k8 in range(HID // L):
                            w_v[row, pl.ds(k8 * L, L)] = (
                                w_v[row, pl.ds(k8 * L, L)] * eb)
                        return cy4

                    lax.fori_loop(0, L, rowscale, 0)
                    return cy3

                lax.fori_loop(0, vps, vec, 0)
                return cy2

            lax.fori_loop(0, nsub, subloop, 0)

            def subscat(j, cy2):
                pltpu.sync_copy(w_v.at[pl.ds(j * SUB, SUB)],
                                S_sh.at[id_v.at[j]], add=True)
                return cy2

            lax.fori_loop(0, nsub, subscat, 0)
            return cy

        lax.fori_loop(0, nch, chunk, 0)
        plsc.subcore_barrier()
        pltpu.sync_copy(d_v, d_out.at[wid])

        @pl.when(s < NS - 1)
        def _():
            pltpu.sync_copy(S_sh.at[pl.ds(my_base, rp_a)],
                            s_out.at[c, pl.ds(my_base, rp_a)])

        @pl.when(s == NS - 1)
        def _():
            pltpu.sync_copy(S_sh.at[pl.ds(my_base, rp_last)],
                            s_out.at[c, pl.ds(my_base, rp_last)])

    return scat_k


def _final_body(s_ref, d_ref, x_ref, gn_ref, bn_ref, out_ref):
    S = s_ref[0] + s_ref[1]
    d = jnp.sum(d_ref[...], axis=0) + 1e-16
    agg = S / d[:, None]
    out_ref[...] = _ln(agg + x_ref[...], gn_ref[...], bn_ref[...])


def _final_ln(s_part, d_part, x, p):
    n = x.shape[0]
    nb = 512
    full = lambda s: pl.BlockSpec(s, lambda i: (0, 0))
    return pl.pallas_call(
        _final_body,
        grid=(pl.cdiv(n, nb),),
        in_specs=[
            pl.BlockSpec((NC, nb, HID), lambda i: (0, i, 0)),
            pl.BlockSpec((NW, nb), lambda i: (0, i)),
            pl.BlockSpec((nb, HID), lambda i: (i, 0)),
            full((1, HID)), full((1, HID)),
        ],
        out_specs=pl.BlockSpec((nb, HID), lambda i: (i, 0)),
        out_shape=jax.ShapeDtypeStruct((n, HID), jnp.float32),
    )(s_part, d_part, x, p['gn'][None], p['bn'][None])


def _block(x, pos3, src, dst, p):
    n = x.shape[0]
    e = src.shape[0]
    feat = _node_precompute(x, p)
    gd, gs, dpx, dpy, dpz = _make_gather(n, e)(feat, pos3, src, dst)
    W, score = _edge_dense(gd, gs, dpx, dpy, dpz, p)
    m_part = _make_segmax(n, e)(score, dst)
    m = _mreduce(m_part)
    s_part, d_part = _make_scatter(n, e)(W, score, dst, m)
    return _final_ln(s_part, d_part, x, p)


def kernel(x, pos, edge_index, params):
    src = edge_index[0]
    dst = edge_index[1]
    pos3 = pos.T.reshape(-1)
    for p in params:
        x = _block(x, pos3, src, dst, p)
    return x


# speedup vs baseline: 6.9320x; 1.2757x over previous
import functools
import math

import jax
import jax.numpy as jnp
from jax import lax
from jax.experimental import pallas as pl
from jax.experimental.pallas import tpu as pltpu
from jax.experimental.pallas import tpu_sc as plsc

HID = 128
EPS = 1e-5
TW = 144
NC = 2
NS = 16
L = 16
NW = NC * NS


def _ln(z, g, b):
    mu = jnp.mean(z, axis=-1, keepdims=True)
    var = jnp.mean((z - mu) ** 2, axis=-1, keepdims=True)
    return (z - mu) * jax.lax.rsqrt(var + EPS) * g + b


def _vgather(x, idx):
    return lax.gather(
        x, idx[:, None],
        lax.GatherDimensionNumbers(
            offset_dims=(), collapsed_slice_dims=(0,), start_index_map=(0,)),
        (1,), mode=lax.GatherScatterMode.PROMISE_IN_BOUNDS)


def _node_body(x_ref, wf_ref, bf_ref, gf_ref, b2f_ref, feat_ref):
    z = jnp.maximum(
        jnp.dot(x_ref[...], wf_ref[...], preferred_element_type=jnp.float32)
        + bf_ref[...], 0.0)
    feat_ref[...] = _ln(z, gf_ref[...], b2f_ref[...])


def _node_precompute(x, p):
    n = x.shape[0]
    blk = 512
    full = lambda s: pl.BlockSpec(s, lambda i: (0, 0))
    return pl.pallas_call(
        _node_body,
        grid=(pl.cdiv(n, blk),),
        in_specs=[
            pl.BlockSpec((blk, HID), lambda i: (i, 0)),
            full((HID, HID)),
            full((1, HID)), full((1, HID)), full((1, HID)),
        ],
        out_specs=pl.BlockSpec((blk, HID), lambda i: (i, 0)),
        out_shape=jax.ShapeDtypeStruct((n, HID), jnp.float32),
    )(x, p['Wf'], p['bf'][None], p['gf'][None], p['b2f'][None])


@functools.cache
def _make_gather(n, e):
    epw = e // NW
    CG = 80
    nch = epw // CG
    rp_a = (n // NS) // 8 * 8
    rp_last = n - (NS - 1) * rp_a
    mesh = plsc.VectorSubcoreMesh(core_axis_name="c", subcore_axis_name="s")

    @functools.partial(
        pl.kernel,
        out_type=[jax.ShapeDtypeStruct((e, HID), jnp.float32),
                  jax.ShapeDtypeStruct((e, HID), jnp.float32),
                  jax.ShapeDtypeStruct((e,), jnp.float32),
                  jax.ShapeDtypeStruct((e,), jnp.float32),
                  jax.ShapeDtypeStruct((e,), jnp.float32)],
        mesh=mesh,
        compiler_params=pltpu.CompilerParams(needs_layout_passes=False),
        scratch_types=[
            pltpu.VMEM((3 * n,), jnp.float32),
            pltpu.VMEM((CG,), jnp.int32),
            pltpu.VMEM((CG,), jnp.int32),
            pltpu.VMEM((CG, HID), jnp.float32),
            pltpu.VMEM((CG, HID), jnp.float32),
            pltpu.VMEM((3, CG), jnp.float32),
            pltpu.SemaphoreType.DMA,
            pltpu.SemaphoreType.DMA,
        ],
    )
    def gather_k(tab_hbm, pos3_hbm, src_hbm, dst_hbm,
                 gd_out, gs_out, dpx_out, dpy_out, dpz_out,
                 pos3_v, idxd_v, idxs_v, gd_v, gs_v, dp3_v,
                 sem1, sem2):
        c = lax.axis_index("c")
        s = lax.axis_index("s")
        wid = s * NC + c
        pltpu.sync_copy(pos3_hbm, pos3_v)
        base_w = wid * epw
        dp_outs = (dpx_out, dpy_out, dpz_out)

        def chunk(i, carry):
            base = base_w + i * CG
            pltpu.sync_copy(dst_hbm.at[pl.ds(base, CG)], idxd_v)
            pltpu.sync_copy(src_hbm.at[pl.ds(base, CG)], idxs_v)
            cp1 = pltpu.async_copy(tab_hbm.at[idxd_v], gd_v, sem1)
            cp2 = pltpu.async_copy(tab_hbm.at[idxs_v], gs_v, sem2)

            def vec(v, carry2):
                ivd = idxd_v[pl.ds(v * L, L)]
                ivs = idxs_v[pl.ds(v * L, L)]
                for comp in range(3):
                    off = jnp.int32(comp * n)
                    d = (plsc.load_gather(pos3_v, [ivd + off])
                         - plsc.load_gather(pos3_v, [ivs + off]))
                    dp3_v[comp, pl.ds(v * L, L)] = d
                return carry2

            lax.fori_loop(0, CG // L, vec, 0)
            cp1.wait()
            cp2.wait()
            pltpu.sync_copy(gd_v, gd_out.at[pl.ds(base, CG)])
            pltpu.sync_copy(gs_v, gs_out.at[pl.ds(base, CG)])
            for comp in range(3):
                pltpu.sync_copy(dp3_v.at[comp], dp_outs[comp].at[pl.ds(base, CG)])
            return carry

        lax.fori_loop(0, nch, chunk, 0)

    return gather_k


def _edge_body(gd_ref, gs_ref, dpx_ref, dpy_ref, dpz_ref,
               wwf_ref, wwp_ref, bw_ref, gw_ref, b2w_ref,
               wp_ref, bp_ref, gp_ref, b2p_ref, wq_ref, bq_ref,
               wk_ref, bk_ref, w_ref, score_ref):
    gd = gd_ref[...]
    gs = gs_ref[...]
    dpc = (dpx_ref[...][:, None], dpy_ref[...][:, None], dpz_ref[...][:, None])
    wwp = wwp_ref[...]
    wp = wp_ref[...]
    tdp = dpc[0] * wwp[0:1] + dpc[1] * wwp[1:2] + dpc[2] * wwp[2:3]
    pdp = dpc[0] * wp[0:1] + dpc[1] * wp[1:2] + dpc[2] * wp[2:3]
    h = (jnp.dot(gd - gs, wwf_ref[...], preferred_element_type=jnp.float32)
         + tdp + bw_ref[...])
    W = _ln(jnp.maximum(h, 0.0), gw_ref[...], b2w_ref[...])
    pe = _ln(jnp.maximum(pdp + bp_ref[...], 0.0), gp_ref[...], b2p_ref[...])
    q = (jnp.dot(gd, wq_ref[...], preferred_element_type=jnp.float32)
         + bq_ref[...] + pe)
    k = jnp.dot(W, wk_ref[...], preferred_element_type=jnp.float32) + bk_ref[...]
    w_ref[...] = W
    score_ref[...] = jnp.sum(q * k, axis=-1) * (1.0 / math.sqrt(float(HID)))


def _edge_dense(gd, gs, dpx, dpy, dpz, p):
    e = gd.shape[0]
    blk = 1024
    full = lambda s: pl.BlockSpec(s, lambda i: (0, 0))
    return pl.pallas_call(
        _edge_body,
        grid=(pl.cdiv(e, blk),),
        in_specs=[
            pl.BlockSpec((blk, HID), lambda i: (i, 0)),
            pl.BlockSpec((blk, HID), lambda i: (i, 0)),
            pl.BlockSpec((blk,), lambda i: (i,)),
            pl.BlockSpec((blk,), lambda i: (i,)),
            pl.BlockSpec((blk,), lambda i: (i,)),
            full((HID, HID)),
            full((8, HID)),
            full((1, HID)), full((1, HID)), full((1, HID)),
            full((8, HID)),
            full((1, HID)), full((1, HID)), full((1, HID)),
            full((HID, HID)),
            full((1, HID)),
            full((HID, HID)),
            full((1, HID)),
        ],
        out_specs=[
            pl.BlockSpec((blk, HID), lambda i: (i, 0)),
            pl.BlockSpec((blk,), lambda i: (i,)),
        ],
        out_shape=[
            jax.ShapeDtypeStruct((e, HID), jnp.float32),
            jax.ShapeDtypeStruct((e,), jnp.float32),
        ],
    )(gd, gs, dpx, dpy, dpz,
      p['Ww'][:HID],
      jnp.zeros((8, HID), jnp.float32).at[:3].set(p['Ww'][HID:]),
      p['bw'][None], p['gw'][None], p['b2w'][None],
      jnp.zeros((8, HID), jnp.float32).at[:3].set(p['Wp']),
      p['bp'][None], p['gp'][None], p['b2p'][None],
      p['Wq'], p['bq'][None], p['Wk'], p['bk'][None])


@functools.cache
def _make_segmax(n, e):
    epw = e // NW
    CH = 2000
    nch = epw // CH
    nvec = CH // L
    mesh = plsc.VectorSubcoreMesh(core_axis_name="c", subcore_axis_name="s")

    @functools.partial(
        pl.kernel,
        out_type=jax.ShapeDtypeStruct((NW, n), jnp.float32),
        mesh=mesh,
        compiler_params=pltpu.CompilerParams(needs_layout_passes=False),
        scratch_types=[
            pltpu.VMEM((n,), jnp.float32),
            pltpu.VMEM((CH,), jnp.float32),
            pltpu.VMEM((CH,), jnp.int32),
        ],
    )
    def segmax_k(score_hbm, dst_hbm, m_out, m_v, sc_v, id_v):
        c = lax.axis_index("c")
        s = lax.axis_index("s")
        wid = s * NC + c
        neg = jnp.full((L,), -jnp.inf, jnp.float32)

        def zi(j, carry):
            m_v[pl.ds(j * L, L)] = neg
            return carry

        lax.fori_loop(0, n // L, zi, 0)
        base_w = wid * epw

        def chunk(i, carry):
            base = base_w + i * CH
            pltpu.sync_copy(score_hbm.at[pl.ds(base, CH)], sc_v)
            pltpu.sync_copy(dst_hbm.at[pl.ds(base, CH)], id_v)

            def vec(v, carry2):
                iv = id_v[pl.ds(v * L, L)]
                sv = sc_v[pl.ds(v * L, L)]
                ks, vs = plsc.sort_key_val(iv, sv)
                iota = lax.iota(jnp.int32, L)
                for sh in (1, 2, 4, 8):
                    pidx = jnp.maximum(iota - sh, 0)
                    valid = (iota >= sh) & (_vgather(ks, pidx) == ks)
                    vs = jnp.maximum(
                        vs, jnp.where(valid, _vgather(vs, pidx), -jnp.inf))
                is_last = (iota == L - 1) | (
                    _vgather(ks, jnp.minimum(iota + 1, L - 1)) != ks)
                cur = plsc.load_gather(m_v, [ks])
                upd = is_last & (vs > cur)
                plsc.store_scatter(m_v, [ks], vs, mask=upd)
                return carry2

            lax.fori_loop(0, nvec, vec, 0)
            return carry

        lax.fori_loop(0, nch, chunk, 0)
        pltpu.sync_copy(m_v, m_out.at[wid])

    return segmax_k


def _mreduce_body(mp_ref, m_ref):
    m_ref[...] = jnp.max(mp_ref[...], axis=0)


def _mreduce(m_part):
    nw, n = m_part.shape
    nb = 1024
    return pl.pallas_call(
        _mreduce_body,
        grid=(pl.cdiv(n, nb),),
        in_specs=[pl.BlockSpec((nw, nb), lambda i: (0, i))],
        out_specs=pl.BlockSpec((nb,), lambda i: (i,)),
        out_shape=jax.ShapeDtypeStruct((n,), jnp.float32),
    )(m_part)


@functools.cache
def _make_scatter(n, e, ncnk):
    ec = e // ncnk
    epw = ec // NW
    CH = 80
    SUB = 80
    nsub = CH // SUB
    nch = epw // CH
    vps = SUB // L
    rp_a = (n // NS) // 8 * 8
    rp_last = n - (NS - 1) * rp_a
    mesh = plsc.VectorSubcoreMesh(core_axis_name="c", subcore_axis_name="s")

    @functools.partial(
        pl.kernel,
        out_type=[jax.ShapeDtypeStruct((NC, n, HID), jnp.float32),
                  jax.ShapeDtypeStruct((NW, n), jnp.float32)],
        mesh=mesh,
        compiler_params=pltpu.CompilerParams(needs_layout_passes=False),
        scratch_types=[
            pltpu.VMEM_SHARED((n, HID), jnp.float32),
            pltpu.VMEM((n,), jnp.float32),
            pltpu.VMEM((n,), jnp.float32),
            pltpu.VMEM((CH, HID), jnp.float32),
            pltpu.VMEM((nsub, SUB), jnp.int32),
            pltpu.VMEM((CH,), jnp.float32),
        ],
    )
    def scat_k(w0, w1, w2, w3, w4, sc0, sc1, sc2, sc3, sc4,
               dst_hbm, m_hbm, s_out, d_out,
               S_sh, m_v, d_v, w_v, id_v, sc_v):
        w_hbms = (w0, w1, w2, w3, w4)
        sc_hbms = (sc0, sc1, sc2, sc3, sc4)
        c = lax.axis_index("c")
        s = lax.axis_index("s")
        wid = s * NC + c
        pltpu.sync_copy(m_hbm, m_v)
        zf = jnp.zeros((L,), jnp.float32)

        def zd(j, cy):
            d_v[pl.ds(j * L, L)] = zf
            return cy

        lax.fori_loop(0, n // L, zd, 0)

        def zw(j, cy):
            for k8 in range(HID // L):
                w_v[j, pl.ds(k8 * L, L)] = zf
            return cy

        lax.fori_loop(0, CH, zw, 0)
        my_base = s * rp_a
        nz_full = rp_a // CH
        nz_rem = rp_a - nz_full * CH
        nz_full_last = rp_last // CH
        nz_rem_last = rp_last - nz_full_last * CH

        def zcopy(j, cy):
            pltpu.sync_copy(w_v, S_sh.at[pl.ds(my_base + j * CH, CH)])
            return cy

        @pl.when(s < NS - 1)
        def _():
            lax.fori_loop(0, nz_full, zcopy, 0)
            if nz_rem:
                pltpu.sync_copy(
                    w_v.at[pl.ds(0, nz_rem)],
                    S_sh.at[pl.ds(my_base + nz_full * CH, nz_rem)])

        @pl.when(s == NS - 1)
        def _():
            lax.fori_loop(0, nz_full_last, zcopy, 0)
            if nz_rem_last:
                pltpu.sync_copy(
                    w_v.at[pl.ds(0, nz_rem_last)],
                    S_sh.at[pl.ds(my_base + nz_full_last * CH, nz_rem_last)])

        plsc.subcore_barrier()
        base_w = wid * epw

        def make_chunk(w_hbm, score_hbm, dst_off):
          def chunk(i, cy):
            base = base_w + i * CH
            pltpu.sync_copy(score_hbm.at[pl.ds(base, CH)], sc_v)

            def sub_idx(j, cy2):
                pltpu.sync_copy(dst_hbm.at[pl.ds(dst_off + base + j * SUB, SUB)],
                                id_v.at[j])
                return cy2

            lax.fori_loop(0, nsub, sub_idx, 0)
            pltpu.sync_copy(w_hbm.at[pl.ds(base, CH)], w_v)

            def subloop(j, cy2):
                def vec(u, cy3):
                    off = j * SUB + u * L
                    iv = id_v[j, pl.ds(u * L, L)]
                    sv = sc_v[pl.ds(off, L)]
                    mg = plsc.load_gather(m_v, [iv])
                    ev = jnp.exp(sv - mg)
                    ks, vs = plsc.sort_key_val(iv, ev)
                    iota = lax.iota(jnp.int32, L)
                    for sh in (1, 2, 4, 8):
                        pidx = jnp.maximum(iota - sh, 0)
                        valid = (iota >= sh) & (_vgather(ks, pidx) == ks)
                        vs = vs + jnp.where(valid, _vgather(vs, pidx), 0.0)
                    is_last = (iota == L - 1) | (
                        _vgather(ks, jnp.minimum(iota + 1, L - 1)) != ks)
                    plsc.addupdate_scatter(d_v, [ks], vs, mask=is_last)

                    def rowscale(r, cy4):
                        eb = _vgather(ev, jnp.full((L,), r, jnp.int32))
                        row = off + r
                        for k8 in range(HID // L):
                            w_v[row, pl.ds(k8 * L, L)] = (
                                w_v[row, pl.ds(k8 * L, L)] * eb)
                        return cy4

                    lax.fori_loop(0, L, rowscale, 0)
                    return cy3

                lax.fori_loop(0, vps, vec, 0)
                return cy2

            lax.fori_loop(0, nsub, subloop, 0)

            def subscat(j, cy2):
                pltpu.sync_copy(w_v.at[pl.ds(j * SUB, SUB)],
                                S_sh.at[id_v.at[j]], add=True)
                return cy2

            lax.fori_loop(0, nsub, subscat, 0)
            return cy
          return chunk

        for cnk in range(ncnk):
            lax.fori_loop(0, nch,
                          make_chunk(w_hbms[cnk], sc_hbms[cnk], cnk * ec), 0)
        plsc.subcore_barrier()
        pltpu.sync_copy(d_v, d_out.at[wid])

        @pl.when(s < NS - 1)
        def _():
            pltpu.sync_copy(S_sh.at[pl.ds(my_base, rp_a)],
                            s_out.at[c, pl.ds(my_base, rp_a)])

        @pl.when(s == NS - 1)
        def _():
            pltpu.sync_copy(S_sh.at[pl.ds(my_base, rp_last)],
                            s_out.at[c, pl.ds(my_base, rp_last)])

    return scat_k


def _final_body(s_ref, d_ref, x_ref, gn_ref, bn_ref, out_ref):
    S = s_ref[0] + s_ref[1]
    d = jnp.sum(d_ref[...], axis=0) + 1e-16
    agg = S / d[:, None]
    out_ref[...] = _ln(agg + x_ref[...], gn_ref[...], bn_ref[...])


def _final_ln(s_part, d_part, x, p):
    n = x.shape[0]
    nb = 512
    full = lambda s: pl.BlockSpec(s, lambda i: (0, 0))
    return pl.pallas_call(
        _final_body,
        grid=(pl.cdiv(n, nb),),
        in_specs=[
            pl.BlockSpec((NC, nb, HID), lambda i: (0, i, 0)),
            pl.BlockSpec((NW, nb), lambda i: (0, i)),
            pl.BlockSpec((nb, HID), lambda i: (i, 0)),
            full((1, HID)), full((1, HID)),
        ],
        out_specs=pl.BlockSpec((nb, HID), lambda i: (i, 0)),
        out_shape=jax.ShapeDtypeStruct((n, HID), jnp.float32),
    )(s_part, d_part, x, p['gn'][None], p['bn'][None])


def _block(x, pos3, src, dst, p):
    n = x.shape[0]
    e = src.shape[0]
    ncnk = 5
    ec = e // ncnk
    feat = _node_precompute(x, p)
    ws, scs, ms = [], [], []
    for c in range(ncnk):
        src_c = lax.slice_in_dim(src, c * ec, (c + 1) * ec)
        dst_c = lax.slice_in_dim(dst, c * ec, (c + 1) * ec)
        gd, gs, dpx, dpy, dpz = _make_gather(n, ec)(feat, pos3, src_c, dst_c)
        w_c, sc_c = _edge_dense(gd, gs, dpx, dpy, dpz, p)
        ms.append(_make_segmax(n, ec)(sc_c, dst_c))
        ws.append(w_c)
        scs.append(sc_c)
    m = _mreduce(jnp.concatenate(ms, axis=0))
    s_part, d_part = _make_scatter(n, e, ncnk)(*ws, *scs, dst, m)
    return _final_ln(s_part, d_part, x, p)


def kernel(x, pos, edge_index, params):
    src = edge_index[0]
    dst = edge_index[1]
    pos3 = pos.T.reshape(-1)
    for p in params:
        x = _block(x, pos3, src, dst, p)
    return x


# speedup vs baseline: 6.9524x; 1.0029x over previous
import functools
import math

import jax
import jax.numpy as jnp
from jax import lax
from jax.experimental import pallas as pl
from jax.experimental.pallas import tpu as pltpu
from jax.experimental.pallas import tpu_sc as plsc

HID = 128
EPS = 1e-5
TW = 144
NC = 2
NS = 16
L = 16
NW = NC * NS


def _ln(z, g, b):
    mu = jnp.mean(z, axis=-1, keepdims=True)
    var = jnp.mean((z - mu) ** 2, axis=-1, keepdims=True)
    return (z - mu) * jax.lax.rsqrt(var + EPS) * g + b


def _vgather(x, idx):
    return lax.gather(
        x, idx[:, None],
        lax.GatherDimensionNumbers(
            offset_dims=(), collapsed_slice_dims=(0,), start_index_map=(0,)),
        (1,), mode=lax.GatherScatterMode.PROMISE_IN_BOUNDS)


def _node_body(x_ref, wf_ref, bf_ref, gf_ref, b2f_ref, feat_ref):
    z = jnp.maximum(
        jnp.dot(x_ref[...], wf_ref[...], preferred_element_type=jnp.float32)
        + bf_ref[...], 0.0)
    feat_ref[...] = _ln(z, gf_ref[...], b2f_ref[...])


def _node_precompute(x, p):
    n = x.shape[0]
    blk = 512
    full = lambda s: pl.BlockSpec(s, lambda i: (0, 0))
    return pl.pallas_call(
        _node_body,
        grid=(pl.cdiv(n, blk),),
        in_specs=[
            pl.BlockSpec((blk, HID), lambda i: (i, 0)),
            full((HID, HID)),
            full((1, HID)), full((1, HID)), full((1, HID)),
        ],
        out_specs=pl.BlockSpec((blk, HID), lambda i: (i, 0)),
        out_shape=jax.ShapeDtypeStruct((n, HID), jnp.float32),
    )(x, p['Wf'], p['bf'][None], p['gf'][None], p['b2f'][None])


@functools.cache
def _make_gather(n, e):
    epw = e // NW
    CG = 80
    nch = epw // CG
    rp_a = (n // NS) // 8 * 8
    rp_last = n - (NS - 1) * rp_a
    mesh = plsc.VectorSubcoreMesh(core_axis_name="c", subcore_axis_name="s")

    @functools.partial(
        pl.kernel,
        out_type=[jax.ShapeDtypeStruct((e, HID), jnp.float32),
                  jax.ShapeDtypeStruct((e, HID), jnp.float32),
                  jax.ShapeDtypeStruct((e,), jnp.float32),
                  jax.ShapeDtypeStruct((e,), jnp.float32),
                  jax.ShapeDtypeStruct((e,), jnp.float32)],
        mesh=mesh,
        compiler_params=pltpu.CompilerParams(needs_layout_passes=False),
        scratch_types=[
            pltpu.VMEM((3 * n,), jnp.float32),
            pltpu.VMEM((epw,), jnp.int32),
            pltpu.VMEM((epw,), jnp.int32),
            pltpu.VMEM((2, CG, HID), jnp.float32),
            pltpu.VMEM((2, CG, HID), jnp.float32),
            pltpu.VMEM((3, CG), jnp.float32),
            pltpu.SemaphoreType.DMA,
            pltpu.SemaphoreType.DMA,
            pltpu.SemaphoreType.DMA,
            pltpu.SemaphoreType.DMA,
        ],
    )
    def gather_k(tab_hbm, pos3_hbm, src_hbm, dst_hbm,
                 gd_out, gs_out, dpx_out, dpy_out, dpz_out,
                 pos3_v, idxd_v, idxs_v, gd_v, gs_v, dp3_v,
                 sem1, sem2, sem3, sem4):
        c = lax.axis_index("c")
        s = lax.axis_index("s")
        wid = s * NC + c
        pltpu.sync_copy(pos3_hbm, pos3_v)
        base_w = wid * epw
        pltpu.sync_copy(dst_hbm.at[pl.ds(base_w, epw)], idxd_v)
        pltpu.sync_copy(src_hbm.at[pl.ds(base_w, epw)], idxs_v)
        dp_outs = (dpx_out, dpy_out, dpz_out)
        sems = ((sem1, sem2), (sem3, sem4))

        def issue(i, b):
            return (pltpu.async_copy(tab_hbm.at[idxd_v.at[pl.ds(i * CG, CG)]],
                                     gd_v.at[b], sems[b][0]),
                    pltpu.async_copy(tab_hbm.at[idxs_v.at[pl.ds(i * CG, CG)]],
                                     gs_v.at[b], sems[b][1]))

        cps = issue(0, 0)
        for i in range(nch):
            b = i % 2
            nxt = None
            if i + 1 < nch:
                nxt = issue(i + 1, 1 - b)
            base = base_w + i * CG

            def vec(v, carry2, i=i):
                ivd = idxd_v[pl.ds(i * CG + v * L, L)]
                ivs = idxs_v[pl.ds(i * CG + v * L, L)]
                for comp in range(3):
                    off = jnp.int32(comp * n)
                    d = (plsc.load_gather(pos3_v, [ivd + off])
                         - plsc.load_gather(pos3_v, [ivs + off]))
                    dp3_v[comp, pl.ds(v * L, L)] = d
                return carry2

            lax.fori_loop(0, CG // L, vec, 0)
            for comp in range(3):
                pltpu.sync_copy(dp3_v.at[comp],
                                dp_outs[comp].at[pl.ds(base, CG)])
            cps[0].wait()
            cps[1].wait()
            pltpu.sync_copy(gd_v.at[b], gd_out.at[pl.ds(base, CG)])
            pltpu.sync_copy(gs_v.at[b], gs_out.at[pl.ds(base, CG)])
            cps = nxt

    return gather_k


def _edge_body(gd_ref, gs_ref, dpx_ref, dpy_ref, dpz_ref,
               wwf_ref, wwp_ref, bw_ref, gw_ref, b2w_ref,
               wp_ref, bp_ref, gp_ref, b2p_ref, wq_ref, bq_ref,
               wk_ref, bk_ref, w_ref, score_ref):
    gd = gd_ref[...]
    gs = gs_ref[...]
    dpc = (dpx_ref[...][:, None], dpy_ref[...][:, None], dpz_ref[...][:, None])
    wwp = wwp_ref[...]
    wp = wp_ref[...]
    tdp = dpc[0] * wwp[0:1] + dpc[1] * wwp[1:2] + dpc[2] * wwp[2:3]
    pdp = dpc[0] * wp[0:1] + dpc[1] * wp[1:2] + dpc[2] * wp[2:3]
    h = (jnp.dot(gd - gs, wwf_ref[...], preferred_element_type=jnp.float32)
         + tdp + bw_ref[...])
    W = _ln(jnp.maximum(h, 0.0), gw_ref[...], b2w_ref[...])
    pe = _ln(jnp.maximum(pdp + bp_ref[...], 0.0), gp_ref[...], b2p_ref[...])
    q = (jnp.dot(gd, wq_ref[...], preferred_element_type=jnp.float32)
         + bq_ref[...] + pe)
    k = jnp.dot(W, wk_ref[...], preferred_element_type=jnp.float32) + bk_ref[...]
    w_ref[...] = W
    score_ref[...] = jnp.sum(q * k, axis=-1) * (1.0 / math.sqrt(float(HID)))


def _edge_dense(gd, gs, dpx, dpy, dpz, p):
    e = gd.shape[0]
    blk = 1024
    full = lambda s: pl.BlockSpec(s, lambda i: (0, 0))
    return pl.pallas_call(
        _edge_body,
        grid=(pl.cdiv(e, blk),),
        in_specs=[
            pl.BlockSpec((blk, HID), lambda i: (i, 0)),
            pl.BlockSpec((blk, HID), lambda i: (i, 0)),
            pl.BlockSpec((blk,), lambda i: (i,)),
            pl.BlockSpec((blk,), lambda i: (i,)),
            pl.BlockSpec((blk,), lambda i: (i,)),
            full((HID, HID)),
            full((8, HID)),
            full((1, HID)), full((1, HID)), full((1, HID)),
            full((8, HID)),
            full((1, HID)), full((1, HID)), full((1, HID)),
            full((HID, HID)),
            full((1, HID)),
            full((HID, HID)),
            full((1, HID)),
        ],
        out_specs=[
            pl.BlockSpec((blk, HID), lambda i: (i, 0)),
            pl.BlockSpec((blk,), lambda i: (i,)),
        ],
        out_shape=[
            jax.ShapeDtypeStruct((e, HID), jnp.float32),
            jax.ShapeDtypeStruct((e,), jnp.float32),
        ],
    )(gd, gs, dpx, dpy, dpz,
      p['Ww'][:HID],
      jnp.zeros((8, HID), jnp.float32).at[:3].set(p['Ww'][HID:]),
      p['bw'][None], p['gw'][None], p['b2w'][None],
      jnp.zeros((8, HID), jnp.float32).at[:3].set(p['Wp']),
      p['bp'][None], p['gp'][None], p['b2p'][None],
      p['Wq'], p['bq'][None], p['Wk'], p['bk'][None])


@functools.cache
def _make_segmax(n, e):
    epw = e // NW
    CH = 2000
    nch = epw // CH
    nvec = CH // L
    mesh = plsc.VectorSubcoreMesh(core_axis_name="c", subcore_axis_name="s")

    @functools.partial(
        pl.kernel,
        out_type=jax.ShapeDtypeStruct((NW, n), jnp.float32),
        mesh=mesh,
        compiler_params=pltpu.CompilerParams(needs_layout_passes=False),
        scratch_types=[
            pltpu.VMEM((n,), jnp.float32),
            pltpu.VMEM((CH,), jnp.float32),
            pltpu.VMEM((CH,), jnp.int32),
        ],
    )
    def segmax_k(score_hbm, dst_hbm, m_out, m_v, sc_v, id_v):
        c = lax.axis_index("c")
        s = lax.axis_index("s")
        wid = s * NC + c
        neg = jnp.full((L,), -jnp.inf, jnp.float32)

        def zi(j, carry):
            m_v[pl.ds(j * L, L)] = neg
            return carry

        lax.fori_loop(0, n // L, zi, 0)
        base_w = wid * epw

        def chunk(i, carry):
            base = base_w + i * CH
            pltpu.sync_copy(score_hbm.at[pl.ds(base, CH)], sc_v)
            pltpu.sync_copy(dst_hbm.at[pl.ds(base, CH)], id_v)

            def vec(v, carry2):
                iv = id_v[pl.ds(v * L, L)]
                sv = sc_v[pl.ds(v * L, L)]
                ks, vs = plsc.sort_key_val(iv, sv)
                iota = lax.iota(jnp.int32, L)
                for sh in (1, 2, 4, 8):
                    pidx = jnp.maximum(iota - sh, 0)
                    valid = (iota >= sh) & (_vgather(ks, pidx) == ks)
                    vs = jnp.maximum(
                        vs, jnp.where(valid, _vgather(vs, pidx), -jnp.inf))
                is_last = (iota == L - 1) | (
                    _vgather(ks, jnp.minimum(iota + 1, L - 1)) != ks)
                cur = plsc.load_gather(m_v, [ks])
                upd = is_last & (vs > cur)
                plsc.store_scatter(m_v, [ks], vs, mask=upd)
                return carry2

            lax.fori_loop(0, nvec, vec, 0)
            return carry

        lax.fori_loop(0, nch, chunk, 0)
        pltpu.sync_copy(m_v, m_out.at[wid])

    return segmax_k


def _mreduce_body(mp_ref, m_ref):
    m_ref[...] = jnp.max(mp_ref[...], axis=0)


def _mreduce(m_part):
    nw, n = m_part.shape
    nb = 1024
    return pl.pallas_call(
        _mreduce_body,
        grid=(pl.cdiv(n, nb),),
        in_specs=[pl.BlockSpec((nw, nb), lambda i: (0, i))],
        out_specs=pl.BlockSpec((nb,), lambda i: (i,)),
        out_shape=jax.ShapeDtypeStruct((n,), jnp.float32),
    )(m_part)


@functools.cache
def _make_scatter(n, e, ncnk):
    ec = e // ncnk
    epw = ec // NW
    CH = 80
    SUB = 80
    nsub = CH // SUB
    nch = epw // CH
    vps = SUB // L
    rp_a = (n // NS) // 8 * 8
    rp_last = n - (NS - 1) * rp_a
    mesh = plsc.VectorSubcoreMesh(core_axis_name="c", subcore_axis_name="s")

    @functools.partial(
        pl.kernel,
        out_type=[jax.ShapeDtypeStruct((NC, n, HID), jnp.float32),
                  jax.ShapeDtypeStruct((NW, n), jnp.float32)],
        mesh=mesh,
        compiler_params=pltpu.CompilerParams(needs_layout_passes=False),
        scratch_types=[
            pltpu.VMEM_SHARED((n, HID), jnp.float32),
            pltpu.VMEM((n,), jnp.float32),
            pltpu.VMEM((n,), jnp.float32),
            pltpu.VMEM((CH, HID), jnp.float32),
            pltpu.VMEM((nsub, SUB), jnp.int32),
            pltpu.VMEM((CH,), jnp.float32),
        ],
    )
    def scat_k(w0, w1, w2, w3, w4, sc0, sc1, sc2, sc3, sc4,
               dst_hbm, m_hbm, s_out, d_out,
               S_sh, m_v, d_v, w_v, id_v, sc_v):
        w_hbms = (w0, w1, w2, w3, w4)
        sc_hbms = (sc0, sc1, sc2, sc3, sc4)
        c = lax.axis_index("c")
        s = lax.axis_index("s")
        wid = s * NC + c
        pltpu.sync_copy(m_hbm, m_v)
        zf = jnp.zeros((L,), jnp.float32)

        def zd(j, cy):
            d_v[pl.ds(j * L, L)] = zf
            return cy

        lax.fori_loop(0, n // L, zd, 0)

        def zw(j, cy):
            for k8 in range(HID // L):
                w_v[j, pl.ds(k8 * L, L)] = zf
            return cy

        lax.fori_loop(0, CH, zw, 0)
        my_base = s * rp_a
        nz_full = rp_a // CH
        nz_rem = rp_a - nz_full * CH
        nz_full_last = rp_last // CH
        nz_rem_last = rp_last - nz_full_last * CH

        def zcopy(j, cy):
            pltpu.sync_copy(w_v, S_sh.at[pl.ds(my_base + j * CH, CH)])
            return cy

        @pl.when(s < NS - 1)
        def _():
            lax.fori_loop(0, nz_full, zcopy, 0)
            if nz_rem:
                pltpu.sync_copy(
                    w_v.at[pl.ds(0, nz_rem)],
                    S_sh.at[pl.ds(my_base + nz_full * CH, nz_rem)])

        @pl.when(s == NS - 1)
        def _():
            lax.fori_loop(0, nz_full_last, zcopy, 0)
            if nz_rem_last:
                pltpu.sync_copy(
                    w_v.at[pl.ds(0, nz_rem_last)],
                    S_sh.at[pl.ds(my_base + nz_full_last * CH, nz_rem_last)])

        plsc.subcore_barrier()
        base_w = wid * epw

        def make_chunk(w_hbm, score_hbm, dst_off):
          def chunk(i, cy):
            base = base_w + i * CH
            pltpu.sync_copy(score_hbm.at[pl.ds(base, CH)], sc_v)

            def sub_idx(j, cy2):
                pltpu.sync_copy(dst_hbm.at[pl.ds(dst_off + base + j * SUB, SUB)],
                                id_v.at[j])
                return cy2

            lax.fori_loop(0, nsub, sub_idx, 0)
            pltpu.sync_copy(w_hbm.at[pl.ds(base, CH)], w_v)

            def subloop(j, cy2):
                def vec(u, cy3):
                    off = j * SUB + u * L
                    iv = id_v[j, pl.ds(u * L, L)]
                    sv = sc_v[pl.ds(off, L)]
                    mg = plsc.load_gather(m_v, [iv])
                    ev = jnp.exp(sv - mg)
                    ks, vs = plsc.sort_key_val(iv, ev)
                    iota = lax.iota(jnp.int32, L)
                    for sh in (1, 2, 4, 8):
                        pidx = jnp.maximum(iota - sh, 0)
                        valid = (iota >= sh) & (_vgather(ks, pidx) == ks)
                        vs = vs + jnp.where(valid, _vgather(vs, pidx), 0.0)
                    is_last = (iota == L - 1) | (
                        _vgather(ks, jnp.minimum(iota + 1, L - 1)) != ks)
                    plsc.addupdate_scatter(d_v, [ks], vs, mask=is_last)

                    def rowscale(r, cy4):
                        eb = _vgather(ev, jnp.full((L,), r, jnp.int32))
                        row = off + r
                        for k8 in range(HID // L):
                            w_v[row, pl.ds(k8 * L, L)] = (
                                w_v[row, pl.ds(k8 * L, L)] * eb)
                        return cy4

                    lax.fori_loop(0, L, rowscale, 0)
                    return cy3

                lax.fori_loop(0, vps, vec, 0)
                return cy2

            lax.fori_loop(0, nsub, subloop, 0)

            def subscat(j, cy2):
                pltpu.sync_copy(w_v.at[pl.ds(j * SUB, SUB)],
                                S_sh.at[id_v.at[j]], add=True)
                return cy2

            lax.fori_loop(0, nsub, subscat, 0)
            return cy
          return chunk

        for cnk in range(ncnk):
            lax.fori_loop(0, nch,
                          make_chunk(w_hbms[cnk], sc_hbms[cnk], cnk * ec), 0)
        plsc.subcore_barrier()
        pltpu.sync_copy(d_v, d_out.at[wid])

        @pl.when(s < NS - 1)
        def _():
            pltpu.sync_copy(S_sh.at[pl.ds(my_base, rp_a)],
                            s_out.at[c, pl.ds(my_base, rp_a)])

        @pl.when(s == NS - 1)
        def _():
            pltpu.sync_copy(S_sh.at[pl.ds(my_base, rp_last)],
                            s_out.at[c, pl.ds(my_base, rp_last)])

    return scat_k


def _final_body(s_ref, d_ref, x_ref, gn_ref, bn_ref, out_ref):
    S = s_ref[0] + s_ref[1]
    d = jnp.sum(d_ref[...], axis=0) + 1e-16
    agg = S / d[:, None]
    out_ref[...] = _ln(agg + x_ref[...], gn_ref[...], bn_ref[...])


def _final_ln(s_part, d_part, x, p):
    n = x.shape[0]
    nb = 512
    full = lambda s: pl.BlockSpec(s, lambda i: (0, 0))
    return pl.pallas_call(
        _final_body,
        grid=(pl.cdiv(n, nb),),
        in_specs=[
            pl.BlockSpec((NC, nb, HID), lambda i: (0, i, 0)),
            pl.BlockSpec((NW, nb), lambda i: (0, i)),
            pl.BlockSpec((nb, HID), lambda i: (i, 0)),
            full((1, HID)), full((1, HID)),
        ],
        out_specs=pl.BlockSpec((nb, HID), lambda i: (i, 0)),
        out_shape=jax.ShapeDtypeStruct((n, HID), jnp.float32),
    )(s_part, d_part, x, p['gn'][None], p['bn'][None])


def _block(x, pos3, src, dst, p):
    n = x.shape[0]
    e = src.shape[0]
    ncnk = 5
    ec = e // ncnk
    feat = _node_precompute(x, p)
    ws, scs, ms = [], [], []
    for c in range(ncnk):
        src_c = lax.slice_in_dim(src, c * ec, (c + 1) * ec)
        dst_c = lax.slice_in_dim(dst, c * ec, (c + 1) * ec)
        gd, gs, dpx, dpy, dpz = _make_gather(n, ec)(feat, pos3, src_c, dst_c)
        w_c, sc_c = _edge_dense(gd, gs, dpx, dpy, dpz, p)
        ms.append(_make_segmax(n, ec)(sc_c, dst_c))
        ws.append(w_c)
        scs.append(sc_c)
    m = _mreduce(jnp.concatenate(ms, axis=0))
    s_part, d_part = _make_scatter(n, e, ncnk)(*ws, *scs, dst, m)
    return _final_ln(s_part, d_part, x, p)


def kernel(x, pos, edge_index, params):
    src = edge_index[0]
    dst = edge_index[1]
    pos3 = pos.T.reshape(-1)
    for p in params:
        x = _block(x, pos3, src, dst, p)
    return x


# speedup vs baseline: 8.1333x; 1.1698x over previous
import functools
import math

import jax
import jax.numpy as jnp
from jax import lax
from jax.experimental import pallas as pl
from jax.experimental.pallas import tpu as pltpu
from jax.experimental.pallas import tpu_sc as plsc

HID = 128
EPS = 1e-5
TW = 144
NC = 2
NS = 16
L = 16
NW = NC * NS


def _ln(z, g, b):
    mu = jnp.mean(z, axis=-1, keepdims=True)
    var = jnp.mean((z - mu) ** 2, axis=-1, keepdims=True)
    return (z - mu) * jax.lax.rsqrt(var + EPS) * g + b


def _vgather(x, idx):
    return lax.gather(
        x, idx[:, None],
        lax.GatherDimensionNumbers(
            offset_dims=(), collapsed_slice_dims=(0,), start_index_map=(0,)),
        (1,), mode=lax.GatherScatterMode.PROMISE_IN_BOUNDS)


def _node_body(x_ref, wf_ref, bf_ref, gf_ref, b2f_ref, feat_ref):
    z = jnp.maximum(
        jnp.dot(x_ref[...], wf_ref[...], preferred_element_type=jnp.float32)
        + bf_ref[...], 0.0)
    feat_ref[...] = _ln(z, gf_ref[...], b2f_ref[...])


def _node_precompute(x, p):
    n = x.shape[0]
    blk = 512
    full = lambda s: pl.BlockSpec(s, lambda i: (0, 0))
    return pl.pallas_call(
        _node_body,
        grid=(pl.cdiv(n, blk),),
        in_specs=[
            pl.BlockSpec((blk, HID), lambda i: (i, 0)),
            full((HID, HID)),
            full((1, HID)), full((1, HID)), full((1, HID)),
        ],
        out_specs=pl.BlockSpec((blk, HID), lambda i: (i, 0)),
        out_shape=jax.ShapeDtypeStruct((n, HID), jnp.float32),
    )(x, p['Wf'], p['bf'][None], p['gf'][None], p['b2f'][None])


@functools.cache
def _make_gather(n, e):
    epw = e // NW
    CG = 80
    nch = epw // CG
    rp_a = (n // NS) // 8 * 8
    rp_last = n - (NS - 1) * rp_a
    mesh = plsc.VectorSubcoreMesh(core_axis_name="c", subcore_axis_name="s")

    @functools.partial(
        pl.kernel,
        out_type=[jax.ShapeDtypeStruct((e, HID), jnp.float32),
                  jax.ShapeDtypeStruct((e, HID), jnp.float32),
                  jax.ShapeDtypeStruct((e,), jnp.float32),
                  jax.ShapeDtypeStruct((e,), jnp.float32),
                  jax.ShapeDtypeStruct((e,), jnp.float32)],
        mesh=mesh,
        compiler_params=pltpu.CompilerParams(needs_layout_passes=False),
        scratch_types=[
            pltpu.VMEM((3 * n,), jnp.float32),
            pltpu.VMEM((epw,), jnp.int32),
            pltpu.VMEM((epw,), jnp.int32),
            pltpu.VMEM((2, CG, HID), jnp.float32),
            pltpu.VMEM((2, CG, HID), jnp.float32),
            pltpu.VMEM((3, CG), jnp.float32),
            pltpu.SemaphoreType.DMA,
            pltpu.SemaphoreType.DMA,
            pltpu.SemaphoreType.DMA,
            pltpu.SemaphoreType.DMA,
        ],
    )
    def gather_k(tab_hbm, pos3_hbm, src_hbm, dst_hbm,
                 gd_out, gs_out, dpx_out, dpy_out, dpz_out,
                 pos3_v, idxd_v, idxs_v, gd_v, gs_v, dp3_v,
                 sem1, sem2, sem3, sem4):
        c = lax.axis_index("c")
        s = lax.axis_index("s")
        wid = s * NC + c
        pltpu.sync_copy(pos3_hbm, pos3_v)
        base_w = wid * epw
        pltpu.sync_copy(dst_hbm.at[pl.ds(base_w, epw)], idxd_v)
        pltpu.sync_copy(src_hbm.at[pl.ds(base_w, epw)], idxs_v)
        dp_outs = (dpx_out, dpy_out, dpz_out)
        sems = ((sem1, sem2), (sem3, sem4))

        def issue(i, b):
            return (pltpu.async_copy(tab_hbm.at[idxd_v.at[pl.ds(i * CG, CG)]],
                                     gd_v.at[b], sems[b][0]),
                    pltpu.async_copy(tab_hbm.at[idxs_v.at[pl.ds(i * CG, CG)]],
                                     gs_v.at[b], sems[b][1]))

        cps = issue(0, 0)
        for i in range(nch):
            b = i % 2
            nxt = None
            if i + 1 < nch:
                nxt = issue(i + 1, 1 - b)
            base = base_w + i * CG

            def vec(v, carry2, i=i):
                ivd = idxd_v[pl.ds(i * CG + v * L, L)]
                ivs = idxs_v[pl.ds(i * CG + v * L, L)]
                for comp in range(3):
                    off = jnp.int32(comp * n)
                    d = (plsc.load_gather(pos3_v, [ivd + off])
                         - plsc.load_gather(pos3_v, [ivs + off]))
                    dp3_v[comp, pl.ds(v * L, L)] = d
                return carry2

            lax.fori_loop(0, CG // L, vec, 0)
            for comp in range(3):
                pltpu.sync_copy(dp3_v.at[comp],
                                dp_outs[comp].at[pl.ds(base, CG)])
            cps[0].wait()
            cps[1].wait()
            pltpu.sync_copy(gd_v.at[b], gd_out.at[pl.ds(base, CG)])
            pltpu.sync_copy(gs_v.at[b], gs_out.at[pl.ds(base, CG)])
            cps = nxt

    return gather_k


def _edge_body(gd_ref, gs_ref, dpx_ref, dpy_ref, dpz_ref,
               wwf_ref, wwp_ref, bw_ref, gw_ref, b2w_ref,
               wp_ref, bp_ref, gp_ref, b2p_ref, wq_ref, bq_ref,
               wk_ref, bk_ref, w_ref, score_ref):
    gd = gd_ref[...]
    gs = gs_ref[...]
    dpc = (dpx_ref[...][:, None], dpy_ref[...][:, None], dpz_ref[...][:, None])
    wwp = wwp_ref[...]
    wp = wp_ref[...]
    tdp = dpc[0] * wwp[0:1] + dpc[1] * wwp[1:2] + dpc[2] * wwp[2:3]
    pdp = dpc[0] * wp[0:1] + dpc[1] * wp[1:2] + dpc[2] * wp[2:3]
    h = (jnp.dot(gd - gs, wwf_ref[...], preferred_element_type=jnp.float32)
         + tdp + bw_ref[...])
    W = _ln(jnp.maximum(h, 0.0), gw_ref[...], b2w_ref[...])
    pe = _ln(jnp.maximum(pdp + bp_ref[...], 0.0), gp_ref[...], b2p_ref[...])
    q = (jnp.dot(gd, wq_ref[...], preferred_element_type=jnp.float32)
         + bq_ref[...] + pe)
    k = jnp.dot(W, wk_ref[...], preferred_element_type=jnp.float32) + bk_ref[...]
    w_ref[...] = W
    score_ref[...] = jnp.sum(q * k, axis=-1) * (1.0 / math.sqrt(float(HID)))


def _edge_dense(gd, gs, dpx, dpy, dpz, p):
    e = gd.shape[0]
    blk = 1024
    full = lambda s: pl.BlockSpec(s, lambda i: (0, 0))
    return pl.pallas_call(
        _edge_body,
        grid=(pl.cdiv(e, blk),),
        in_specs=[
            pl.BlockSpec((blk, HID), lambda i: (i, 0)),
            pl.BlockSpec((blk, HID), lambda i: (i, 0)),
            pl.BlockSpec((blk,), lambda i: (i,)),
            pl.BlockSpec((blk,), lambda i: (i,)),
            pl.BlockSpec((blk,), lambda i: (i,)),
            full((HID, HID)),
            full((8, HID)),
            full((1, HID)), full((1, HID)), full((1, HID)),
            full((8, HID)),
            full((1, HID)), full((1, HID)), full((1, HID)),
            full((HID, HID)),
            full((1, HID)),
            full((HID, HID)),
            full((1, HID)),
        ],
        out_specs=[
            pl.BlockSpec((blk, HID), lambda i: (i, 0)),
            pl.BlockSpec((blk,), lambda i: (i,)),
        ],
        out_shape=[
            jax.ShapeDtypeStruct((e, HID), jnp.float32),
            jax.ShapeDtypeStruct((e,), jnp.float32),
        ],
    )(gd, gs, dpx, dpy, dpz,
      p['Ww'][:HID],
      jnp.zeros((8, HID), jnp.float32).at[:3].set(p['Ww'][HID:]),
      p['bw'][None], p['gw'][None], p['b2w'][None],
      jnp.zeros((8, HID), jnp.float32).at[:3].set(p['Wp']),
      p['bp'][None], p['gp'][None], p['b2p'][None],
      p['Wq'], p['bq'][None], p['Wk'], p['bk'][None])


@functools.cache
def _make_segmax(n, e):
    epw = e // NW
    CH = 2000
    nch = epw // CH
    nvec = CH // L
    mesh = plsc.VectorSubcoreMesh(core_axis_name="c", subcore_axis_name="s")

    @functools.partial(
        pl.kernel,
        out_type=jax.ShapeDtypeStruct((NW, n), jnp.float32),
        mesh=mesh,
        compiler_params=pltpu.CompilerParams(needs_layout_passes=False),
        scratch_types=[
            pltpu.VMEM((n,), jnp.float32),
            pltpu.VMEM((CH,), jnp.float32),
            pltpu.VMEM((CH,), jnp.int32),
        ],
    )
    def segmax_k(score_hbm, dst_hbm, m_out, m_v, sc_v, id_v):
        c = lax.axis_index("c")
        s = lax.axis_index("s")
        wid = s * NC + c
        neg = jnp.full((L,), -jnp.inf, jnp.float32)

        def zi(j, carry):
            m_v[pl.ds(j * L, L)] = neg
            return carry

        lax.fori_loop(0, n // L, zi, 0)
        base_w = wid * epw

        def chunk(i, carry):
            base = base_w + i * CH
            pltpu.sync_copy(score_hbm.at[pl.ds(base, CH)], sc_v)
            pltpu.sync_copy(dst_hbm.at[pl.ds(base, CH)], id_v)

            def vec(v, carry2):
                iv = id_v[pl.ds(v * L, L)]
                sv = sc_v[pl.ds(v * L, L)]
                ks, vs = plsc.sort_key_val(iv, sv)
                iota = lax.iota(jnp.int32, L)
                for sh in (1, 2, 4, 8):
                    pidx = jnp.maximum(iota - sh, 0)
                    valid = (iota >= sh) & (_vgather(ks, pidx) == ks)
                    vs = jnp.maximum(
                        vs, jnp.where(valid, _vgather(vs, pidx), -jnp.inf))
                is_last = (iota == L - 1) | (
                    _vgather(ks, jnp.minimum(iota + 1, L - 1)) != ks)
                cur = plsc.load_gather(m_v, [ks])
                upd = is_last & (vs > cur)
                plsc.store_scatter(m_v, [ks], vs, mask=upd)
                return carry2

            lax.fori_loop(0, nvec, vec, 0)
            return carry

        lax.fori_loop(0, nch, chunk, 0)
        pltpu.sync_copy(m_v, m_out.at[wid])

    return segmax_k


def _mreduce_body(mp_ref, m_ref):
    m_ref[...] = jnp.max(mp_ref[...], axis=0)


def _mreduce(m_part):
    nw, n = m_part.shape
    nb = 1024
    return pl.pallas_call(
        _mreduce_body,
        grid=(pl.cdiv(n, nb),),
        in_specs=[pl.BlockSpec((nw, nb), lambda i: (0, i))],
        out_specs=pl.BlockSpec((nb,), lambda i: (i,)),
        out_shape=jax.ShapeDtypeStruct((n,), jnp.float32),
    )(m_part)


@functools.cache
def _make_scatter(n, e, ncnk):
    ec = e // ncnk
    epw = ec // NW
    CH = 80
    SUB = 80
    nsub = CH // SUB
    nch = epw // CH
    vps = SUB // L
    rp_a = (n // NS) // 8 * 8
    rp_last = n - (NS - 1) * rp_a
    mesh = plsc.VectorSubcoreMesh(core_axis_name="c", subcore_axis_name="s")

    @functools.partial(
        pl.kernel,
        out_type=[jax.ShapeDtypeStruct((NC, n, HID), jnp.float32),
                  jax.ShapeDtypeStruct((NW, n), jnp.float32)],
        mesh=mesh,
        compiler_params=pltpu.CompilerParams(needs_layout_passes=False),
        scratch_types=[
            pltpu.VMEM_SHARED((n, HID), jnp.float32),
            pltpu.VMEM((n,), jnp.float32),
            pltpu.VMEM((n,), jnp.float32),
            pltpu.VMEM((2, CH, HID), jnp.float32),
            pltpu.VMEM((epw,), jnp.int32),
            pltpu.VMEM((epw,), jnp.float32),
            pltpu.SemaphoreType.DMA,
            pltpu.SemaphoreType.DMA,
            pltpu.SemaphoreType.DMA,
            pltpu.SemaphoreType.DMA,
        ],
    )
    def scat_k(w0, w1, w2, w3, w4, sc0, sc1, sc2, sc3, sc4,
               dst_hbm, m_hbm, s_out, d_out,
               S_sh, m_v, d_v, w_v, id_v, sc_v,
               sem_r0, sem_r1, sem_s0, sem_s1):
        w_hbms = (w0, w1, w2, w3, w4)
        sc_hbms = (sc0, sc1, sc2, sc3, sc4)
        c = lax.axis_index("c")
        s = lax.axis_index("s")
        wid = s * NC + c
        pltpu.sync_copy(m_hbm, m_v)
        zf = jnp.zeros((L,), jnp.float32)

        def zd(j, cy):
            d_v[pl.ds(j * L, L)] = zf
            return cy

        lax.fori_loop(0, n // L, zd, 0)

        def zw(j, cy):
            for k8 in range(HID // L):
                w_v[0, j, pl.ds(k8 * L, L)] = zf
            return cy

        lax.fori_loop(0, CH, zw, 0)
        my_base = s * rp_a
        nz_full = rp_a // CH
        nz_rem = rp_a - nz_full * CH
        nz_full_last = rp_last // CH
        nz_rem_last = rp_last - nz_full_last * CH

        def zcopy(j, cy):
            pltpu.sync_copy(w_v.at[0], S_sh.at[pl.ds(my_base + j * CH, CH)])
            return cy

        @pl.when(s < NS - 1)
        def _():
            lax.fori_loop(0, nz_full, zcopy, 0)
            if nz_rem:
                pltpu.sync_copy(
                    w_v.at[0].at[pl.ds(0, nz_rem)],
                    S_sh.at[pl.ds(my_base + nz_full * CH, nz_rem)])

        @pl.when(s == NS - 1)
        def _():
            lax.fori_loop(0, nz_full_last, zcopy, 0)
            if nz_rem_last:
                pltpu.sync_copy(
                    w_v.at[0].at[pl.ds(0, nz_rem_last)],
                    S_sh.at[pl.ds(my_base + nz_full_last * CH, nz_rem_last)])

        plsc.subcore_barrier()
        base_w = wid * epw
        rsems = (sem_r0, sem_r1)
        ssems = (sem_s0, sem_s1)
        def vwork(i, b):
            def vec(u, cy3):
                off = i * CH + u * L
                iv = id_v[pl.ds(off, L)]
                sv = sc_v[pl.ds(off, L)]
                mg = plsc.load_gather(m_v, [iv])
                ev = jnp.exp(sv - mg)
                ks, vs = plsc.sort_key_val(iv, ev)
                iota = lax.iota(jnp.int32, L)
                for sh in (1, 2, 4, 8):
                    pidx = jnp.maximum(iota - sh, 0)
                    valid = (iota >= sh) & (_vgather(ks, pidx) == ks)
                    vs = vs + jnp.where(valid, _vgather(vs, pidx), 0.0)
                is_last = (iota == L - 1) | (
                    _vgather(ks, jnp.minimum(iota + 1, L - 1)) != ks)
                plsc.addupdate_scatter(d_v, [ks], vs, mask=is_last)

                def rowscale(r, cy4):
                    eb = _vgather(ev, jnp.full((L,), r, jnp.int32))
                    row = u * L + r
                    for k8 in range(HID // L):
                        w_v[b, row, pl.ds(k8 * L, L)] = (
                            w_v[b, row, pl.ds(k8 * L, L)] * eb)
                    return cy4

                lax.fori_loop(0, L, rowscale, 0)
                return cy3

            lax.fori_loop(0, vps, vec, 0)

        def one_chunk(w_hbm, i, b):
            rd = pltpu.async_copy(
                w_hbm.at[pl.ds(base_w + i * CH, CH)], w_v.at[b], rsems[b])
            return rd

        def run_chunk(i, b, rd):
            rd.wait()
            vwork(i, b)
            return pltpu.async_copy(
                w_v.at[b], S_sh.at[id_v.at[pl.ds(i * CH, CH)]],
                ssems[b], add=True)

        for cnk in range(ncnk):
            w_hbm = w_hbms[cnk]
            score_hbm = sc_hbms[cnk]
            pltpu.sync_copy(score_hbm.at[pl.ds(base_w, epw)], sc_v)
            pltpu.sync_copy(dst_hbm.at[pl.ds(cnk * ec + base_w, epw)], id_v)

            def pair(t, cy):
                i0 = 2 * t
                rd0 = one_chunk(w_hbm, i0, 0)
                rd1 = one_chunk(w_hbm, i0 + 1, 1)
                sc0 = run_chunk(i0, 0, rd0)
                sc1 = run_chunk(i0 + 1, 1, rd1)
                sc0.wait()
                sc1.wait()
                return cy

            lax.fori_loop(0, nch // 2, pair, 0)
            for i in range(nch // 2 * 2, nch):
                rd = one_chunk(w_hbm, i, 0)
                sc0 = run_chunk(i, 0, rd)
                sc0.wait()
        plsc.subcore_barrier()
        pltpu.sync_copy(d_v, d_out.at[wid])

        @pl.when(s < NS - 1)
        def _():
            pltpu.sync_copy(S_sh.at[pl.ds(my_base, rp_a)],
                            s_out.at[c, pl.ds(my_base, rp_a)])

        @pl.when(s == NS - 1)
        def _():
            pltpu.sync_copy(S_sh.at[pl.ds(my_base, rp_last)],
                            s_out.at[c, pl.ds(my_base, rp_last)])

    return scat_k


def _final_body(s_ref, d_ref, x_ref, gn_ref, bn_ref, out_ref):
    S = s_ref[0] + s_ref[1]
    d = jnp.sum(d_ref[...], axis=0) + 1e-16
    agg = S / d[:, None]
    out_ref[...] = _ln(agg + x_ref[...], gn_ref[...], bn_ref[...])


def _final_ln(s_part, d_part, x, p):
    n = x.shape[0]
    nb = 512
    full = lambda s: pl.BlockSpec(s, lambda i: (0, 0))
    return pl.pallas_call(
        _final_body,
        grid=(pl.cdiv(n, nb),),
        in_specs=[
            pl.BlockSpec((NC, nb, HID), lambda i: (0, i, 0)),
            pl.BlockSpec((NW, nb), lambda i: (0, i)),
            pl.BlockSpec((nb, HID), lambda i: (i, 0)),
            full((1, HID)), full((1, HID)),
        ],
        out_specs=pl.BlockSpec((nb, HID), lambda i: (i, 0)),
        out_shape=jax.ShapeDtypeStruct((n, HID), jnp.float32),
    )(s_part, d_part, x, p['gn'][None], p['bn'][None])


def _block(x, pos3, src, dst, p):
    n = x.shape[0]
    e = src.shape[0]
    ncnk = 5
    ec = e // ncnk
    feat = _node_precompute(x, p)
    ws, scs, ms = [], [], []
    for c in range(ncnk):
        src_c = lax.slice_in_dim(src, c * ec, (c + 1) * ec)
        dst_c = lax.slice_in_dim(dst, c * ec, (c + 1) * ec)
        gd, gs, dpx, dpy, dpz = _make_gather(n, ec)(feat, pos3, src_c, dst_c)
        w_c, sc_c = _edge_dense(gd, gs, dpx, dpy, dpz, p)
        ms.append(_make_segmax(n, ec)(sc_c, dst_c))
        ws.append(w_c)
        scs.append(sc_c)
    m = _mreduce(jnp.concatenate(ms, axis=0))
    s_part, d_part = _make_scatter(n, e, ncnk)(*ws, *scs, dst, m)
    return _final_ln(s_part, d_part, x, p)


def kernel(x, pos, edge_index, params):
    src = edge_index[0]
    dst = edge_index[1]
    pos3 = pos.T.reshape(-1)
    for p in params:
        x = _block(x, pos3, src, dst, p)
    return x


# speedup vs baseline: 8.1358x; 1.0003x over previous
import functools
import math

import jax
import jax.numpy as jnp
from jax import lax
from jax.experimental import pallas as pl
from jax.experimental.pallas import tpu as pltpu
from jax.experimental.pallas import tpu_sc as plsc

HID = 128
EPS = 1e-5
TW = 144
NC = 2
NS = 16
L = 16
NW = NC * NS


def _ln(z, g, b):
    mu = jnp.mean(z, axis=-1, keepdims=True)
    var = jnp.mean((z - mu) ** 2, axis=-1, keepdims=True)
    return (z - mu) * jax.lax.rsqrt(var + EPS) * g + b


def _vgather(x, idx):
    return lax.gather(
        x, idx[:, None],
        lax.GatherDimensionNumbers(
            offset_dims=(), collapsed_slice_dims=(0,), start_index_map=(0,)),
        (1,), mode=lax.GatherScatterMode.PROMISE_IN_BOUNDS)


def _node_body(x_ref, wf_ref, bf_ref, gf_ref, b2f_ref, feat_ref):
    z = jnp.maximum(
        jnp.dot(x_ref[...], wf_ref[...], preferred_element_type=jnp.float32)
        + bf_ref[...], 0.0)
    feat_ref[...] = _ln(z, gf_ref[...], b2f_ref[...])


def _node_precompute(x, p):
    n = x.shape[0]
    blk = 512
    full = lambda s: pl.BlockSpec(s, lambda i: (0, 0))
    return pl.pallas_call(
        _node_body,
        grid=(pl.cdiv(n, blk),),
        in_specs=[
            pl.BlockSpec((blk, HID), lambda i: (i, 0)),
            full((HID, HID)),
            full((1, HID)), full((1, HID)), full((1, HID)),
        ],
        out_specs=pl.BlockSpec((blk, HID), lambda i: (i, 0)),
        out_shape=jax.ShapeDtypeStruct((n, HID), jnp.float32),
    )(x, p['Wf'], p['bf'][None], p['gf'][None], p['b2f'][None])


@functools.cache
def _make_gather(n, e):
    epw = e // NW
    CG = 80
    nch = epw // CG
    rp_a = (n // NS) // 8 * 8
    rp_last = n - (NS - 1) * rp_a
    mesh = plsc.VectorSubcoreMesh(core_axis_name="c", subcore_axis_name="s")

    @functools.partial(
        pl.kernel,
        out_type=[jax.ShapeDtypeStruct((e, HID), jnp.float32),
                  jax.ShapeDtypeStruct((e, HID), jnp.float32),
                  jax.ShapeDtypeStruct((e,), jnp.float32),
                  jax.ShapeDtypeStruct((e,), jnp.float32),
                  jax.ShapeDtypeStruct((e,), jnp.float32)],
        mesh=mesh,
        compiler_params=pltpu.CompilerParams(needs_layout_passes=False),
        scratch_types=[
            pltpu.VMEM((3 * n,), jnp.float32),
            pltpu.VMEM((epw,), jnp.int32),
            pltpu.VMEM((epw,), jnp.int32),
            pltpu.VMEM((2, CG, HID), jnp.float32),
            pltpu.VMEM((2, CG, HID), jnp.float32),
            pltpu.VMEM((3, CG), jnp.float32),
            pltpu.SemaphoreType.DMA,
            pltpu.SemaphoreType.DMA,
            pltpu.SemaphoreType.DMA,
            pltpu.SemaphoreType.DMA,
        ],
    )
    def gather_k(tab_hbm, pos3_hbm, src_hbm, dst_hbm,
                 gd_out, gs_out, dpx_out, dpy_out, dpz_out,
                 pos3_v, idxd_v, idxs_v, gd_v, gs_v, dp3_v,
                 sem1, sem2, sem3, sem4):
        c = lax.axis_index("c")
        s = lax.axis_index("s")
        wid = s * NC + c
        pltpu.sync_copy(pos3_hbm, pos3_v)
        base_w = wid * epw
        pltpu.sync_copy(dst_hbm.at[pl.ds(base_w, epw)], idxd_v)
        pltpu.sync_copy(src_hbm.at[pl.ds(base_w, epw)], idxs_v)
        dp_outs = (dpx_out, dpy_out, dpz_out)
        sems = ((sem1, sem2), (sem3, sem4))

        def issue(i, b):
            return (pltpu.async_copy(tab_hbm.at[idxd_v.at[pl.ds(i * CG, CG)]],
                                     gd_v.at[b], sems[b][0]),
                    pltpu.async_copy(tab_hbm.at[idxs_v.at[pl.ds(i * CG, CG)]],
                                     gs_v.at[b], sems[b][1]))

        cps = issue(0, 0)
        for i in range(nch):
            b = i % 2
            nxt = None
            if i + 1 < nch:
                nxt = issue(i + 1, 1 - b)
            base = base_w + i * CG

            def vec(v, carry2, i=i):
                ivd = idxd_v[pl.ds(i * CG + v * L, L)]
                ivs = idxs_v[pl.ds(i * CG + v * L, L)]
                for comp in range(3):
                    off = jnp.int32(comp * n)
                    d = (plsc.load_gather(pos3_v, [ivd + off])
                         - plsc.load_gather(pos3_v, [ivs + off]))
                    dp3_v[comp, pl.ds(v * L, L)] = d
                return carry2

            lax.fori_loop(0, CG // L, vec, 0)
            for comp in range(3):
                pltpu.sync_copy(dp3_v.at[comp],
                                dp_outs[comp].at[pl.ds(base, CG)])
            cps[0].wait()
            cps[1].wait()
            pltpu.sync_copy(gd_v.at[b], gd_out.at[pl.ds(base, CG)])
            pltpu.sync_copy(gs_v.at[b], gs_out.at[pl.ds(base, CG)])
            cps = nxt

    return gather_k


def _edge_body(gd_ref, gs_ref, dpx_ref, dpy_ref, dpz_ref,
               wwf_ref, wwp_ref, bw_ref, gw_ref, b2w_ref,
               wp_ref, bp_ref, gp_ref, b2p_ref, wq_ref, bq_ref,
               wk_ref, bk_ref, w_ref, score_ref):
    gd = gd_ref[...]
    gs = gs_ref[...]
    dpc = (dpx_ref[...][:, None], dpy_ref[...][:, None], dpz_ref[...][:, None])
    wwp = wwp_ref[...]
    wp = wp_ref[...]
    tdp = dpc[0] * wwp[0:1] + dpc[1] * wwp[1:2] + dpc[2] * wwp[2:3]
    pdp = dpc[0] * wp[0:1] + dpc[1] * wp[1:2] + dpc[2] * wp[2:3]
    h = (jnp.dot(gd - gs, wwf_ref[...], preferred_element_type=jnp.float32,
                precision=lax.Precision.DEFAULT)
         + tdp + bw_ref[...])
    W = _ln(jnp.maximum(h, 0.0), gw_ref[...], b2w_ref[...])
    pe = _ln(jnp.maximum(pdp + bp_ref[...], 0.0), gp_ref[...], b2p_ref[...])
    q = (jnp.dot(gd, wq_ref[...], preferred_element_type=jnp.float32,
                precision=lax.Precision.DEFAULT)
         + bq_ref[...] + pe)
    k = jnp.dot(W, wk_ref[...], preferred_element_type=jnp.float32,
                precision=lax.Precision.DEFAULT) + bk_ref[...]
    w_ref[...] = W
    score_ref[...] = jnp.sum(q * k, axis=-1) * (1.0 / math.sqrt(float(HID)))


def _edge_dense(gd, gs, dpx, dpy, dpz, p):
    e = gd.shape[0]
    blk = 1024
    full = lambda s: pl.BlockSpec(s, lambda i: (0, 0))
    return pl.pallas_call(
        _edge_body,
        grid=(pl.cdiv(e, blk),),
        in_specs=[
            pl.BlockSpec((blk, HID), lambda i: (i, 0)),
            pl.BlockSpec((blk, HID), lambda i: (i, 0)),
            pl.BlockSpec((blk,), lambda i: (i,)),
            pl.BlockSpec((blk,), lambda i: (i,)),
            pl.BlockSpec((blk,), lambda i: (i,)),
            full((HID, HID)),
            full((8, HID)),
            full((1, HID)), full((1, HID)), full((1, HID)),
            full((8, HID)),
            full((1, HID)), full((1, HID)), full((1, HID)),
            full((HID, HID)),
            full((1, HID)),
            full((HID, HID)),
            full((1, HID)),
        ],
        out_specs=[
            pl.BlockSpec((blk, HID), lambda i: (i, 0)),
            pl.BlockSpec((blk,), lambda i: (i,)),
        ],
        out_shape=[
            jax.ShapeDtypeStruct((e, HID), jnp.float32),
            jax.ShapeDtypeStruct((e,), jnp.float32),
        ],
    )(gd, gs, dpx, dpy, dpz,
      p['Ww'][:HID],
      jnp.zeros((8, HID), jnp.float32).at[:3].set(p['Ww'][HID:]),
      p['bw'][None], p['gw'][None], p['b2w'][None],
      jnp.zeros((8, HID), jnp.float32).at[:3].set(p['Wp']),
      p['bp'][None], p['gp'][None], p['b2p'][None],
      p['Wq'], p['bq'][None], p['Wk'], p['bk'][None])


@functools.cache
def _make_segmax(n, e):
    epw = e // NW
    CH = 2000
    nch = epw // CH
    nvec = CH // L
    mesh = plsc.VectorSubcoreMesh(core_axis_name="c", subcore_axis_name="s")

    @functools.partial(
        pl.kernel,
        out_type=jax.ShapeDtypeStruct((NW, n), jnp.float32),
        mesh=mesh,
        compiler_params=pltpu.CompilerParams(needs_layout_passes=False),
        scratch_types=[
            pltpu.VMEM((n,), jnp.float32),
            pltpu.VMEM((CH,), jnp.float32),
            pltpu.VMEM((CH,), jnp.int32),
        ],
    )
    def segmax_k(score_hbm, dst_hbm, m_out, m_v, sc_v, id_v):
        c = lax.axis_index("c")
        s = lax.axis_index("s")
        wid = s * NC + c
        neg = jnp.full((L,), -jnp.inf, jnp.float32)

        def zi(j, carry):
            m_v[pl.ds(j * L, L)] = neg
            return carry

        lax.fori_loop(0, n // L, zi, 0)
        base_w = wid * epw

        def chunk(i, carry):
            base = base_w + i * CH
            pltpu.sync_copy(score_hbm.at[pl.ds(base, CH)], sc_v)
            pltpu.sync_copy(dst_hbm.at[pl.ds(base, CH)], id_v)

            def vec(v, carry2):
                iv = id_v[pl.ds(v * L, L)]
                sv = sc_v[pl.ds(v * L, L)]
                ks, vs = plsc.sort_key_val(iv, sv)
                iota = lax.iota(jnp.int32, L)
                for sh in (1, 2, 4, 8):
                    pidx = jnp.maximum(iota - sh, 0)
                    valid = (iota >= sh) & (_vgather(ks, pidx) == ks)
                    vs = jnp.maximum(
                        vs, jnp.where(valid, _vgather(vs, pidx), -jnp.inf))
                is_last = (iota == L - 1) | (
                    _vgather(ks, jnp.minimum(iota + 1, L - 1)) != ks)
                cur = plsc.load_gather(m_v, [ks])
                upd = is_last & (vs > cur)
                plsc.store_scatter(m_v, [ks], vs, mask=upd)
                return carry2

            lax.fori_loop(0, nvec, vec, 0)
            return carry

        lax.fori_loop(0, nch, chunk, 0)
        pltpu.sync_copy(m_v, m_out.at[wid])

    return segmax_k


def _mreduce_body(mp_ref, m_ref):
    m_ref[...] = jnp.max(mp_ref[...], axis=0)


def _mreduce(m_part):
    nw, n = m_part.shape
    nb = 1024
    return pl.pallas_call(
        _mreduce_body,
        grid=(pl.cdiv(n, nb),),
        in_specs=[pl.BlockSpec((nw, nb), lambda i: (0, i))],
        out_specs=pl.BlockSpec((nb,), lambda i: (i,)),
        out_shape=jax.ShapeDtypeStruct((n,), jnp.float32),
    )(m_part)


@functools.cache
def _make_scatter(n, e, ncnk):
    ec = e // ncnk
    epw = ec // NW
    CH = 80
    SUB = 80
    nsub = CH // SUB
    nch = epw // CH
    vps = SUB // L
    rp_a = (n // NS) // 8 * 8
    rp_last = n - (NS - 1) * rp_a
    mesh = plsc.VectorSubcoreMesh(core_axis_name="c", subcore_axis_name="s")

    @functools.partial(
        pl.kernel,
        out_type=[jax.ShapeDtypeStruct((NC, n, HID), jnp.float32),
                  jax.ShapeDtypeStruct((NW, n), jnp.float32)],
        mesh=mesh,
        compiler_params=pltpu.CompilerParams(needs_layout_passes=False),
        scratch_types=[
            pltpu.VMEM_SHARED((n, HID), jnp.float32),
            pltpu.VMEM((n,), jnp.float32),
            pltpu.VMEM((n,), jnp.float32),
            pltpu.VMEM((2, CH, HID), jnp.float32),
            pltpu.VMEM((epw,), jnp.int32),
            pltpu.VMEM((epw,), jnp.float32),
            pltpu.SemaphoreType.DMA,
            pltpu.SemaphoreType.DMA,
            pltpu.SemaphoreType.DMA,
            pltpu.SemaphoreType.DMA,
        ],
    )
    def scat_k(w0, w1, w2, w3, w4, sc0, sc1, sc2, sc3, sc4,
               dst_hbm, m_hbm, s_out, d_out,
               S_sh, m_v, d_v, w_v, id_v, sc_v,
               sem_r0, sem_r1, sem_s0, sem_s1):
        w_hbms = (w0, w1, w2, w3, w4)
        sc_hbms = (sc0, sc1, sc2, sc3, sc4)
        c = lax.axis_index("c")
        s = lax.axis_index("s")
        wid = s * NC + c
        pltpu.sync_copy(m_hbm, m_v)
        zf = jnp.zeros((L,), jnp.float32)

        def zd(j, cy):
            d_v[pl.ds(j * L, L)] = zf
            return cy

        lax.fori_loop(0, n // L, zd, 0)

        def zw(j, cy):
            for k8 in range(HID // L):
                w_v[0, j, pl.ds(k8 * L, L)] = zf
            return cy

        lax.fori_loop(0, CH, zw, 0)
        my_base = s * rp_a
        nz_full = rp_a // CH
        nz_rem = rp_a - nz_full * CH
        nz_full_last = rp_last // CH
        nz_rem_last = rp_last - nz_full_last * CH

        def zcopy(j, cy):
            pltpu.sync_copy(w_v.at[0], S_sh.at[pl.ds(my_base + j * CH, CH)])
            return cy

        @pl.when(s < NS - 1)
        def _():
            lax.fori_loop(0, nz_full, zcopy, 0)
            if nz_rem:
                pltpu.sync_copy(
                    w_v.at[0].at[pl.ds(0, nz_rem)],
                    S_sh.at[pl.ds(my_base + nz_full * CH, nz_rem)])

        @pl.when(s == NS - 1)
        def _():
            lax.fori_loop(0, nz_full_last, zcopy, 0)
            if nz_rem_last:
                pltpu.sync_copy(
                    w_v.at[0].at[pl.ds(0, nz_rem_last)],
                    S_sh.at[pl.ds(my_base + nz_full_last * CH, nz_rem_last)])

        plsc.subcore_barrier()
        base_w = wid * epw
        rsems = (sem_r0, sem_r1)
        ssems = (sem_s0, sem_s1)
        def vwork(i, b):
            def vec(u, cy3):
                off = i * CH + u * L
                iv = id_v[pl.ds(off, L)]
                sv = sc_v[pl.ds(off, L)]
                mg = plsc.load_gather(m_v, [iv])
                ev = jnp.exp(sv - mg)
                ks, vs = plsc.sort_key_val(iv, ev)
                iota = lax.iota(jnp.int32, L)
                for sh in (1, 2, 4, 8):
                    pidx = jnp.maximum(iota - sh, 0)
                    valid = (iota >= sh) & (_vgather(ks, pidx) == ks)
                    vs = vs + jnp.where(valid, _vgather(vs, pidx), 0.0)
                is_last = (iota == L - 1) | (
                    _vgather(ks, jnp.minimum(iota + 1, L - 1)) != ks)
                plsc.addupdate_scatter(d_v, [ks], vs, mask=is_last)

                def rowscale(r, cy4):
                    eb = _vgather(ev, jnp.full((L,), r, jnp.int32))
                    row = u * L + r
                    for k8 in range(HID // L):
                        w_v[b, row, pl.ds(k8 * L, L)] = (
                            w_v[b, row, pl.ds(k8 * L, L)] * eb)
                    return cy4

                lax.fori_loop(0, L, rowscale, 0)
                return cy3

            lax.fori_loop(0, vps, vec, 0)

        def one_chunk(w_hbm, i, b):
            rd = pltpu.async_copy(
                w_hbm.at[pl.ds(base_w + i * CH, CH)], w_v.at[b], rsems[b])
            return rd

        def run_chunk(i, b, rd):
            rd.wait()
            vwork(i, b)
            return pltpu.async_copy(
                w_v.at[b], S_sh.at[id_v.at[pl.ds(i * CH, CH)]],
                ssems[b], add=True)

        for cnk in range(ncnk):
            w_hbm = w_hbms[cnk]
            score_hbm = sc_hbms[cnk]
            pltpu.sync_copy(score_hbm.at[pl.ds(base_w, epw)], sc_v)
            pltpu.sync_copy(dst_hbm.at[pl.ds(cnk * ec + base_w, epw)], id_v)

            def pair(t, cy):
                i0 = 2 * t
                rd0 = one_chunk(w_hbm, i0, 0)
                rd1 = one_chunk(w_hbm, i0 + 1, 1)
                sc0 = run_chunk(i0, 0, rd0)
                sc1 = run_chunk(i0 + 1, 1, rd1)
                sc0.wait()
                sc1.wait()
                return cy

            lax.fori_loop(0, nch // 2, pair, 0)
            for i in range(nch // 2 * 2, nch):
                rd = one_chunk(w_hbm, i, 0)
                sc0 = run_chunk(i, 0, rd)
                sc0.wait()
        plsc.subcore_barrier()
        pltpu.sync_copy(d_v, d_out.at[wid])

        @pl.when(s < NS - 1)
        def _():
            pltpu.sync_copy(S_sh.at[pl.ds(my_base, rp_a)],
                            s_out.at[c, pl.ds(my_base, rp_a)])

        @pl.when(s == NS - 1)
        def _():
            pltpu.sync_copy(S_sh.at[pl.ds(my_base, rp_last)],
                            s_out.at[c, pl.ds(my_base, rp_last)])

    return scat_k


def _final_body(s_ref, d_ref, x_ref, gn_ref, bn_ref, out_ref):
    S = s_ref[0] + s_ref[1]
    d = jnp.sum(d_ref[...], axis=0) + 1e-16
    agg = S / d[:, None]
    out_ref[...] = _ln(agg + x_ref[...], gn_ref[...], bn_ref[...])


def _final_ln(s_part, d_part, x, p):
    n = x.shape[0]
    nb = 512
    full = lambda s: pl.BlockSpec(s, lambda i: (0, 0))
    return pl.pallas_call(
        _final_body,
        grid=(pl.cdiv(n, nb),),
        in_specs=[
            pl.BlockSpec((NC, nb, HID), lambda i: (0, i, 0)),
            pl.BlockSpec((NW, nb), lambda i: (0, i)),
            pl.BlockSpec((nb, HID), lambda i: (i, 0)),
            full((1, HID)), full((1, HID)),
        ],
        out_specs=pl.BlockSpec((nb, HID), lambda i: (i, 0)),
        out_shape=jax.ShapeDtypeStruct((n, HID), jnp.float32),
    )(s_part, d_part, x, p['gn'][None], p['bn'][None])


def _block(x, pos3, src, dst, p):
    n = x.shape[0]
    e = src.shape[0]
    ncnk = 5
    ec = e // ncnk
    feat = _node_precompute(x, p)
    ws, scs, ms = [], [], []
    for c in range(ncnk):
        src_c = lax.slice_in_dim(src, c * ec, (c + 1) * ec)
        dst_c = lax.slice_in_dim(dst, c * ec, (c + 1) * ec)
        gd, gs, dpx, dpy, dpz = _make_gather(n, ec)(feat, pos3, src_c, dst_c)
        w_c, sc_c = _edge_dense(gd, gs, dpx, dpy, dpz, p)
        ms.append(_make_segmax(n, ec)(sc_c, dst_c))
        ws.append(w_c)
        scs.append(sc_c)
    m = _mreduce(jnp.concatenate(ms, axis=0))
    s_part, d_part = _make_scatter(n, e, ncnk)(*ws, *scs, dst, m)
    return _final_ln(s_part, d_part, x, p)


def kernel(x, pos, edge_index, params):
    src = edge_index[0]
    dst = edge_index[1]
    pos3 = pos.T.reshape(-1)
    for p in params:
        x = _block(x, pos3, src, dst, p)
    return x
